# Initial kernel scaffold; baseline (speedup 1.0000x reference)
import jax
import jax.numpy as jnp
from jax.experimental import pallas as pl


def kernel(x, edge_index, W0, as0, ad0, W1, as1, ad1, gWih, gWhh, gbih, gbhh, s_w, out_W):
    raise NotImplementedError("write your pallas kernel here")


# speedup vs baseline: 1.6253x; 1.6253x over previous
import functools

import jax
import jax.numpy as jnp
from jax import lax
from jax.experimental import pallas as pl
from jax.experimental.pallas import tpu as pltpu

N = 10000
E = 160000
IN = 256
H = 128
HEADS = 2
HID = 256
OUT = 256

_GRU_B = 2000


def _dotT(a, b):
    return lax.dot_general(a, b, (((1,), (1,)), ((), ())),
                           preferred_element_type=jnp.float32)


def _gru_out_body(h1_ref, h2_ref, gWih_ref, gWhh_ref, gbih_ref, gbhh_ref,
                  outW_ref, o_ref):
    h1 = h1_ref[...]
    h2 = h2_ref[...]

    def step(xs, h, Wih, Whh, bih, bhh, with_h):
        gi = _dotT(xs, Wih) + bih
        if with_h:
            gh = _dotT(h, Whh) + bhh
        else:
            gh = jnp.broadcast_to(bhh, gi.shape)
        i_r, i_z, i_n = gi[:, :HID], gi[:, HID:2 * HID], gi[:, 2 * HID:]
        h_r, h_z, h_n = gh[:, :HID], gh[:, HID:2 * HID], gh[:, 2 * HID:]
        r = jax.nn.sigmoid(i_r + h_r)
        zg = jax.nn.sigmoid(i_z + h_z)
        n = jnp.tanh(i_n + r * h_n)
        if with_h:
            return (1.0 - zg) * n + zg * h
        return (1.0 - zg) * n

    Wih_f = gWih_ref[0]
    Whh_f = gWhh_ref[0]
    Wih_b = gWih_ref[1]
    Whh_b = gWhh_ref[1]
    bih_f = gbih_ref[0]
    bhh_f = gbhh_ref[0]
    bih_b = gbih_ref[1]
    bhh_b = gbhh_ref[1]

    f0 = step(h1, None, Wih_f, Whh_f, bih_f, bhh_f, False)
    f1 = step(h2, f0, Wih_f, Whh_f, bih_f, bhh_f, True)
    b0 = step(h2, None, Wih_b, Whh_b, bih_b, bhh_b, False)
    b1 = step(h1, b0, Wih_b, Whh_b, bih_b, bhh_b, True)

    left = f0 + f1
    right = b0 + b1
    outW = outW_ref[...]
    o_ref[...] = _dotT(left, outW[:, :HID]) + _dotT(right, outW[:, HID:])


def _gru_out(h1, h2, gWih, gWhh, gbih, gbhh, out_W):
    grid = (N // _GRU_B,)
    full = lambda shape: pl.BlockSpec(shape, lambda i: (0,) * len(shape))
    return pl.pallas_call(
        _gru_out_body,
        grid=grid,
        in_specs=[
            pl.BlockSpec((_GRU_B, HID), lambda i: (i, 0)),
            pl.BlockSpec((_GRU_B, HID), lambda i: (i, 0)),
            full((2, 3 * HID, HID)),
            full((2, 3 * HID, HID)),
            full((2, 3 * HID)),
            full((2, 3 * HID)),
            full((OUT, 2 * HID)),
        ],
        out_specs=pl.BlockSpec((_GRU_B, OUT), lambda i: (i, 0)),
        out_shape=jax.ShapeDtypeStruct((N, OUT), jnp.float32),
    )(h1, h2, gWih, gWhh, gbih, gbhh, out_W)


def _gat_layer(h, src, dst, W, a_s, a_d):
    outs = []
    for i in range(HEADS):
        z = h @ W[i].T
        es = z @ a_s[i]
        ed = z @ a_d[i]
        c = jnp.maximum(jnp.max(es) + jnp.max(ed), 0.0)
        e = es[src] + ed[dst]
        e = jnp.where(e > 0, e, 0.2 * e)
        a = jnp.exp(e - c)
        denom = jax.ops.segment_sum(a, dst, num_segments=N)
        denom = jnp.where(denom > 0, denom, 1.0)
        num = jax.ops.segment_sum(a[:, None] * z[src], dst, num_segments=N)
        outs.append(num / denom[:, None])
    return jnp.concatenate(outs, axis=-1)


def kernel(x, edge_index, W0, as0, ad0, W1, as1, ad1, gWih, gWhh, gbih, gbhh,
           s_w, out_W):
    src = edge_index[0]
    dst = edge_index[1]
    h1 = jax.nn.relu(_gat_layer(x, src, dst, W0, as0, ad0))
    h2 = jax.nn.relu(_gat_layer(h1, src, dst, W1, as1, ad1))
    return _gru_out(h1, h2, gWih, gWhh, gbih, gbhh, out_W)


# speedup vs baseline: 15.8557x; 9.7553x over previous
import functools

import jax
import jax.numpy as jnp
from jax import lax
from jax.experimental import pallas as pl
from jax.experimental.pallas import tpu as pltpu
from jax.experimental.pallas import tpu_sc as plsc

N = 10000
E = 160000
IN = 256
H = 128
HEADS = 2
HID = 256
OUT = 256

NP = 10240
CHUNK = 256
NSUB = CHUNK // 128
EPAD = 163840
NCHUNK = EPAD // (16 * CHUNK)

_B = 2000


def _dotT(a, b):
    return lax.dot_general(a, b, (((1,), (1,)), ((), ())),
                           preferred_element_type=jnp.float32)


def _proj_body(hin_ref, W_ref, as_ref, ad_ref, z_ref, es_ref, ed_ref):
    h = pl.program_id(0)
    z = _dotT(hin_ref[...], W_ref[h])
    z_ref[...] = z
    es_ref[...] = (z @ as_ref[h])[:, None]
    ed_ref[...] = (z @ ad_ref[h])[:, None]


def _proj(hin, W, a_s, a_d):
    grid = (HEADS, N // _B)
    full = lambda shape: pl.BlockSpec(shape, lambda h, i: (0,) * len(shape))
    return pl.pallas_call(
        _proj_body,
        grid=grid,
        in_specs=[
            pl.BlockSpec((_B, IN), lambda h, i: (i, 0)),
            full((HEADS, H, IN)),
            full((HEADS, H)),
            full((HEADS, H)),
        ],
        out_specs=[
            pl.BlockSpec((_B, H), lambda h, i: (h * (N // _B) + i, 0)),
            pl.BlockSpec((_B, 1), lambda h, i: (h * (N // _B) + i, 0)),
            pl.BlockSpec((_B, 1), lambda h, i: (h * (N // _B) + i, 0)),
        ],
        out_shape=[
            jax.ShapeDtypeStruct((HEADS * N, H), jnp.float32),
            jax.ShapeDtypeStruct((HEADS * N, 1), jnp.float32),
            jax.ShapeDtypeStruct((HEADS * N, 1), jnp.float32),
        ],
    )(hin, W, a_s, a_d)


def _norm_proj_body(accA_ref, accB_ref, denA_ref, denB_ref, W_ref, as_ref,
                    ad_ref, hcat_ref, z_ref, es_ref, ed_ref):
    hA = jax.nn.relu(accA_ref[...] / denA_ref[...])
    hB = jax.nn.relu(accB_ref[...] / denB_ref[...])
    hcat = jnp.concatenate([hA, hB], axis=1)
    hcat_ref[...] = hcat
    h = pl.program_id(0)
    z = _dotT(hcat, W_ref[h])
    z_ref[...] = z
    es_ref[...] = (z @ as_ref[h])[:, None]
    ed_ref[...] = (z @ ad_ref[h])[:, None]


def _norm_proj(accA, accB, denA, denB, W, a_s, a_d):
    grid = (HEADS, N // _B)
    full = lambda shape: pl.BlockSpec(shape, lambda h, i: (0,) * len(shape))
    return pl.pallas_call(
        _norm_proj_body,
        grid=grid,
        in_specs=[
            pl.BlockSpec((_B, H), lambda h, i: (i, 0)),
            pl.BlockSpec((_B, H), lambda h, i: (i, 0)),
            pl.BlockSpec((_B, 1), lambda h, i: (i, 0)),
            pl.BlockSpec((_B, 1), lambda h, i: (i, 0)),
            full((HEADS, H, 2 * H)),
            full((HEADS, H)),
            full((HEADS, H)),
        ],
        out_specs=[
            pl.BlockSpec((_B, 2 * H), lambda h, i: (i, 0)),
            pl.BlockSpec((_B, H), lambda h, i: (h * (N // _B) + i, 0)),
            pl.BlockSpec((_B, 1), lambda h, i: (h * (N // _B) + i, 0)),
            pl.BlockSpec((_B, 1), lambda h, i: (h * (N // _B) + i, 0)),
        ],
        out_shape=[
            jax.ShapeDtypeStruct((N, 2 * H), jnp.float32),
            jax.ShapeDtypeStruct((HEADS * N, H), jnp.float32),
            jax.ShapeDtypeStruct((HEADS * N, 1), jnp.float32),
            jax.ShapeDtypeStruct((HEADS * N, 1), jnp.float32),
        ],
    )(accA, accB, denA, denB, W, a_s, a_d)


def _gru_out_body(h1_ref, accA_ref, accB_ref, denA_ref, denB_ref, gWih_ref,
                  gWhh_ref, gbih_ref, gbhh_ref, outW_ref, o_ref):
    h1 = h1_ref[...]
    hA = jax.nn.relu(accA_ref[...] / denA_ref[...])
    hB = jax.nn.relu(accB_ref[...] / denB_ref[...])
    h2 = jnp.concatenate([hA, hB], axis=1)

    def step(xs, h, Wih, Whh, bih, bhh, with_h):
        gi = _dotT(xs, Wih) + bih
        if with_h:
            gh = _dotT(h, Whh) + bhh
        else:
            gh = jnp.broadcast_to(bhh, gi.shape)
        i_r, i_z, i_n = gi[:, :HID], gi[:, HID:2 * HID], gi[:, 2 * HID:]
        h_r, h_z, h_n = gh[:, :HID], gh[:, HID:2 * HID], gh[:, 2 * HID:]
        r = jax.nn.sigmoid(i_r + h_r)
        zg = jax.nn.sigmoid(i_z + h_z)
        n = jnp.tanh(i_n + r * h_n)
        if with_h:
            return (1.0 - zg) * n + zg * h
        return (1.0 - zg) * n

    f0 = step(h1, None, gWih_ref[0], gWhh_ref[0], gbih_ref[0], gbhh_ref[0],
              False)
    f1 = step(h2, f0, gWih_ref[0], gWhh_ref[0], gbih_ref[0], gbhh_ref[0],
              True)
    b0 = step(h2, None, gWih_ref[1], gWhh_ref[1], gbih_ref[1], gbhh_ref[1],
              False)
    b1 = step(h1, b0, gWih_ref[1], gWhh_ref[1], gbih_ref[1], gbhh_ref[1],
              True)

    outW = outW_ref[...]
    o_ref[...] = (_dotT(f0 + f1, outW[:, :HID]) +
                  _dotT(b0 + b1, outW[:, HID:]))


def _gru_out(h1, accA, accB, denA, denB, gWih, gWhh, gbih, gbhh, out_W):
    grid = (N // _B,)
    full = lambda shape: pl.BlockSpec(shape, lambda i: (0,) * len(shape))
    return pl.pallas_call(
        _gru_out_body,
        grid=grid,
        in_specs=[
            pl.BlockSpec((_B, HID), lambda i: (i, 0)),
            pl.BlockSpec((_B, H), lambda i: (i, 0)),
            pl.BlockSpec((_B, H), lambda i: (i, 0)),
            pl.BlockSpec((_B, 1), lambda i: (i, 0)),
            pl.BlockSpec((_B, 1), lambda i: (i, 0)),
            full((2, 3 * HID, HID)),
            full((2, 3 * HID, HID)),
            full((2, 3 * HID)),
            full((2, 3 * HID)),
            full((OUT, 2 * HID)),
        ],
        out_specs=pl.BlockSpec((_B, OUT), lambda i: (i, 0)),
        out_shape=jax.ShapeDtypeStruct((N, OUT), jnp.float32),
    )(h1, accA, accB, denA, denB, gWih, gWhh, gbih, gbhh, out_W)


def _sc_edge_body(z_hbm, esn_hbm, edn_hbm, srcl_hbm, dstl_hbm, cv_hbm,
                  acc_hbm, den_hbm,
                  zrows, src_v, dst_v, dstg_v, a_v, es_c, ed_c, cv_l,
                  acc_s, den_s, sem):
    c = lax.axis_index("c")
    s = lax.axis_index("s")
    cN = c * N

    zero16 = jnp.zeros((16,), jnp.float32)

    def zero_zrows(r, _):
        for k in range(8):
            zrows[r, pl.ds(k * 16, 16)] = zero16
        return 0

    lax.fori_loop(0, CHUNK, zero_zrows, 0)

    def zero_av(i, _):
        a_v[pl.ds(i * 16, 16)] = zero16
        return 0

    lax.fori_loop(0, CHUNK // 16, zero_av, 0)

    row0 = s * 640
    pltpu.sync_copy(zrows.at[pl.ds(0, 256)], acc_s.at[pl.ds(row0, 256)])
    pltpu.sync_copy(zrows.at[pl.ds(0, 256)], acc_s.at[pl.ds(row0 + 256, 256)])
    pltpu.sync_copy(zrows.at[pl.ds(0, 128)], acc_s.at[pl.ds(row0 + 512, 128)])
    pltpu.sync_copy(a_v.at[pl.ds(0, 256)], den_s.at[pl.ds(row0, 256)])
    pltpu.sync_copy(a_v.at[pl.ds(0, 256)], den_s.at[pl.ds(row0 + 256, 256)])
    pltpu.sync_copy(a_v.at[pl.ds(0, 128)], den_s.at[pl.ds(row0 + 512, 128)])

    pltpu.sync_copy(cv_hbm.at[pl.ds(c * 16, 16)], cv_l)

    plsc.subcore_barrier()

    cv = cv_l[...]

    def chunk_body(g, _):
        chunk_id = g * 16 + s
        row = chunk_id * NSUB
        pltpu.sync_copy(srcl_hbm.at[pl.ds(row, NSUB)], src_v)
        pltpu.sync_copy(dstl_hbm.at[pl.ds(row, NSUB)], dst_v)

        for j in range(NSUB):
            for i in range(8):
                off = i * 16
                sv = src_v[j, pl.ds(off, 16)]
                src_v[j, pl.ds(off, 16)] = sv + cN
                dv = dst_v[j, pl.ds(off, 16)]
                dstg_v[j, pl.ds(off, 16)] = jnp.minimum(dv, N - 1) + cN

        for j in range(NSUB):
            pltpu.async_copy(esn_hbm.at[src_v.at[j]],
                             es_c.at[pl.ds(j * 128, 128)], sem).wait()
            pltpu.async_copy(edn_hbm.at[dstg_v.at[j]],
                             ed_c.at[pl.ds(j * 128, 128)], sem).wait()

        for j in range(NSUB):
            for i in range(8):
                off = j * 128 + i * 16
                e = es_c[pl.ds(off, 16)] + ed_c[pl.ds(off, 16)]
                e = jnp.where(e > 0, e, 0.2 * e)
                a_v[pl.ds(off, 16)] = jnp.exp(e - cv)

        for j in range(NSUB):
            pltpu.sync_copy(a_v.at[pl.ds(j * 128, 128)],
                            den_s.at[dst_v.at[j]], add=True)

        for j in range(NSUB):
            pltpu.async_copy(z_hbm.at[src_v.at[j]],
                             zrows.at[pl.ds(j * 128, 128)], sem).wait()

        def scale_rows16(t, _):
            a16 = a_v[pl.ds(t * 16, 16)]
            for j in range(16):
                r = t * 16 + j
                a_sc = a16[j]
                for k in range(8):
                    zrows[r, pl.ds(k * 16, 16)] = (
                        zrows[r, pl.ds(k * 16, 16)] * a_sc)
            return 0

        lax.fori_loop(0, CHUNK // 16, scale_rows16, 0)

        for j in range(NSUB):
            pltpu.sync_copy(zrows.at[pl.ds(j * 128, 128)],
                            acc_s.at[dst_v.at[j]], add=True)
        return 0

    lax.fori_loop(0, NCHUNK, chunk_body, 0)

    plsc.subcore_barrier()

    pltpu.sync_copy(acc_s.at[pl.ds(row0, 640)],
                    acc_hbm.at[pl.ds(c * NP + row0, 640)])
    pltpu.sync_copy(den_s.at[pl.ds(row0, 640)],
                    den_hbm.at[pl.ds(c * NP + row0, 640)])


def _sc_edge(z, esn, edn, srcl, dstl, cvec):
    mesh = plsc.VectorSubcoreMesh(core_axis_name="c", subcore_axis_name="s")
    fn = pl.kernel(
        _sc_edge_body,
        mesh=mesh,
        compiler_params=pltpu.CompilerParams(needs_layout_passes=False),
        out_type=[
            jax.ShapeDtypeStruct((HEADS * NP, H), jnp.float32),
            jax.ShapeDtypeStruct((HEADS * NP,), jnp.float32),
        ],
        scratch_types=[
            pltpu.VMEM((CHUNK, H), jnp.float32),
            pltpu.VMEM((NSUB, 128), jnp.int32),
            pltpu.VMEM((NSUB, 128), jnp.int32),
            pltpu.VMEM((NSUB, 128), jnp.int32),
            pltpu.VMEM((CHUNK,), jnp.float32),
            pltpu.VMEM((CHUNK,), jnp.float32),
            pltpu.VMEM((CHUNK,), jnp.float32),
            pltpu.VMEM((16,), jnp.float32),
            pltpu.VMEM_SHARED((NP, H), jnp.float32),
            pltpu.VMEM_SHARED((NP,), jnp.float32),
            pltpu.SemaphoreType.DMA,
        ],
    )
    return fn(z, esn, edn, srcl, dstl, cvec)


def _prep_scores(es, ed):
    es2 = es.reshape(HEADS, N)
    ed2 = ed.reshape(HEADS, N)
    c2 = jax.nn.relu(jnp.max(es2, axis=1) + jnp.max(ed2, axis=1))
    cvec = jnp.repeat(c2, 16)
    return es.reshape(-1), ed.reshape(-1), cvec


def _split_heads(acc, den):
    accA = acc[:N]
    accB = acc[NP:NP + N]
    denA = jnp.maximum(den[:N], 1e-38)[:, None]
    denB = jnp.maximum(den[NP:NP + N], 1e-38)[:, None]
    return accA, accB, denA, denB


def kernel(x, edge_index, W0, as0, ad0, W1, as1, ad1, gWih, gWhh, gbih, gbhh,
           s_w, out_W):
    src = edge_index[0]
    dst = edge_index[1]
    srcl = jnp.pad(src, (0, EPAD - E)).reshape(-1, 128)
    dstl = jnp.pad(dst, (0, EPAD - E), constant_values=N).reshape(-1, 128)

    z0, es0, ed0 = _proj(x, W0, as0, ad0)
    esn0, edn0, cv0 = _prep_scores(es0, ed0)
    acc0, den0 = _sc_edge(z0, esn0, edn0, srcl, dstl, cv0)
    a0A, a0B, d0A, d0B = _split_heads(acc0, den0)

    h1, z1, es1, ed1 = _norm_proj(a0A, a0B, d0A, d0B, W1, as1, ad1)
    esn1, edn1, cv1 = _prep_scores(es1, ed1)
    acc1, den1 = _sc_edge(z1, esn1, edn1, srcl, dstl, cv1)
    a1A, a1B, d1A, d1B = _split_heads(acc1, den1)

    return _gru_out(h1, a1A, a1B, d1A, d1B, gWih, gWhh, gbih, gbhh, out_W)


# speedup vs baseline: 20.4675x; 1.2909x over previous
import functools

import jax
import jax.numpy as jnp
from jax import lax
from jax.experimental import pallas as pl
from jax.experimental.pallas import tpu as pltpu
from jax.experimental.pallas import tpu_sc as plsc

N = 10000
E = 160000
IN = 256
H = 128
HEADS = 2
HID = 256
OUT = 256

NP = 10240
CHUNK = 256
NSUB = CHUNK // 128
EPAD = 163840
NCHUNK = EPAD // (16 * CHUNK)

_B = 2000


def _dotT(a, b):
    return lax.dot_general(a, b, (((1,), (1,)), ((), ())),
                           preferred_element_type=jnp.float32)


def _proj_body(hin_ref, W_ref, as_ref, ad_ref, z_ref, es_ref, ed_ref):
    h = pl.program_id(0)
    z = _dotT(hin_ref[...], W_ref[h])
    z_ref[...] = z
    es_ref[...] = (z @ as_ref[h])[:, None]
    ed_ref[...] = (z @ ad_ref[h])[:, None]


def _proj(hin, W, a_s, a_d):
    grid = (HEADS, N // _B)
    full = lambda shape: pl.BlockSpec(shape, lambda h, i: (0,) * len(shape))
    return pl.pallas_call(
        _proj_body,
        grid=grid,
        in_specs=[
            pl.BlockSpec((_B, IN), lambda h, i: (i, 0)),
            full((HEADS, H, IN)),
            full((HEADS, H)),
            full((HEADS, H)),
        ],
        out_specs=[
            pl.BlockSpec((_B, H), lambda h, i: (h * (N // _B) + i, 0)),
            pl.BlockSpec((_B, 1), lambda h, i: (h * (N // _B) + i, 0)),
            pl.BlockSpec((_B, 1), lambda h, i: (h * (N // _B) + i, 0)),
        ],
        out_shape=[
            jax.ShapeDtypeStruct((HEADS * N, H), jnp.float32),
            jax.ShapeDtypeStruct((HEADS * N, 1), jnp.float32),
            jax.ShapeDtypeStruct((HEADS * N, 1), jnp.float32),
        ],
    )(hin, W, a_s, a_d)


def _norm_proj_body(accA_ref, accB_ref, denA_ref, denB_ref, W_ref, as_ref,
                    ad_ref, hcat_ref, z_ref, es_ref, ed_ref):
    hA = jax.nn.relu(accA_ref[...] / denA_ref[...])
    hB = jax.nn.relu(accB_ref[...] / denB_ref[...])
    hcat = jnp.concatenate([hA, hB], axis=1)
    hcat_ref[...] = hcat
    h = pl.program_id(0)
    z = _dotT(hcat, W_ref[h])
    z_ref[...] = z
    es_ref[...] = (z @ as_ref[h])[:, None]
    ed_ref[...] = (z @ ad_ref[h])[:, None]


def _norm_proj(accA, accB, denA, denB, W, a_s, a_d):
    grid = (HEADS, N // _B)
    full = lambda shape: pl.BlockSpec(shape, lambda h, i: (0,) * len(shape))
    return pl.pallas_call(
        _norm_proj_body,
        grid=grid,
        in_specs=[
            pl.BlockSpec((_B, H), lambda h, i: (i, 0)),
            pl.BlockSpec((_B, H), lambda h, i: (i, 0)),
            pl.BlockSpec((_B, 1), lambda h, i: (i, 0)),
            pl.BlockSpec((_B, 1), lambda h, i: (i, 0)),
            full((HEADS, H, 2 * H)),
            full((HEADS, H)),
            full((HEADS, H)),
        ],
        out_specs=[
            pl.BlockSpec((_B, 2 * H), lambda h, i: (i, 0)),
            pl.BlockSpec((_B, H), lambda h, i: (h * (N // _B) + i, 0)),
            pl.BlockSpec((_B, 1), lambda h, i: (h * (N // _B) + i, 0)),
            pl.BlockSpec((_B, 1), lambda h, i: (h * (N // _B) + i, 0)),
        ],
        out_shape=[
            jax.ShapeDtypeStruct((N, 2 * H), jnp.float32),
            jax.ShapeDtypeStruct((HEADS * N, H), jnp.float32),
            jax.ShapeDtypeStruct((HEADS * N, 1), jnp.float32),
            jax.ShapeDtypeStruct((HEADS * N, 1), jnp.float32),
        ],
    )(accA, accB, denA, denB, W, a_s, a_d)


def _gru_out_body(h1_ref, accA_ref, accB_ref, denA_ref, denB_ref, gWih_ref,
                  gWhh_ref, gbih_ref, gbhh_ref, outW_ref, o_ref):
    h1 = h1_ref[...]
    hA = jax.nn.relu(accA_ref[...] / denA_ref[...])
    hB = jax.nn.relu(accB_ref[...] / denB_ref[...])
    h2 = jnp.concatenate([hA, hB], axis=1)

    def step(xs, h, Wih, Whh, bih, bhh, with_h):
        gi = _dotT(xs, Wih) + bih
        if with_h:
            gh = _dotT(h, Whh) + bhh
        else:
            gh = jnp.broadcast_to(bhh, gi.shape)
        i_r, i_z, i_n = gi[:, :HID], gi[:, HID:2 * HID], gi[:, 2 * HID:]
        h_r, h_z, h_n = gh[:, :HID], gh[:, HID:2 * HID], gh[:, 2 * HID:]
        r = jax.nn.sigmoid(i_r + h_r)
        zg = jax.nn.sigmoid(i_z + h_z)
        n = jnp.tanh(i_n + r * h_n)
        if with_h:
            return (1.0 - zg) * n + zg * h
        return (1.0 - zg) * n

    f0 = step(h1, None, gWih_ref[0], gWhh_ref[0], gbih_ref[0], gbhh_ref[0],
              False)
    f1 = step(h2, f0, gWih_ref[0], gWhh_ref[0], gbih_ref[0], gbhh_ref[0],
              True)
    b0 = step(h2, None, gWih_ref[1], gWhh_ref[1], gbih_ref[1], gbhh_ref[1],
              False)
    b1 = step(h1, b0, gWih_ref[1], gWhh_ref[1], gbih_ref[1], gbhh_ref[1],
              True)

    outW = outW_ref[...]
    o_ref[...] = (_dotT(f0 + f1, outW[:, :HID]) +
                  _dotT(b0 + b1, outW[:, HID:]))


def _gru_out(h1, accA, accB, denA, denB, gWih, gWhh, gbih, gbhh, out_W):
    grid = (N // _B,)
    full = lambda shape: pl.BlockSpec(shape, lambda i: (0,) * len(shape))
    return pl.pallas_call(
        _gru_out_body,
        grid=grid,
        in_specs=[
            pl.BlockSpec((_B, HID), lambda i: (i, 0)),
            pl.BlockSpec((_B, H), lambda i: (i, 0)),
            pl.BlockSpec((_B, H), lambda i: (i, 0)),
            pl.BlockSpec((_B, 1), lambda i: (i, 0)),
            pl.BlockSpec((_B, 1), lambda i: (i, 0)),
            full((2, 3 * HID, HID)),
            full((2, 3 * HID, HID)),
            full((2, 3 * HID)),
            full((2, 3 * HID)),
            full((OUT, 2 * HID)),
        ],
        out_specs=pl.BlockSpec((_B, OUT), lambda i: (i, 0)),
        out_shape=jax.ShapeDtypeStruct((N, OUT), jnp.float32),
    )(h1, accA, accB, denA, denB, gWih, gWhh, gbih, gbhh, out_W)


def _sc_edge_body(z_hbm, esn_hbm, edn_hbm, srcl_hbm, dstl_hbm, cv_hbm,
                  acc_hbm, den_hbm,
                  zrows, src_v, dst_v, dstg_v, a_v, es_c, ed_c, cv_l,
                  acc_s, den_s, sem):
    c = lax.axis_index("c")
    s = lax.axis_index("s")
    cN = c * N

    zero16 = jnp.zeros((16,), jnp.float32)

    def zero_zrows(r, _):
        for k in range(8):
            zrows[r, pl.ds(k * 16, 16)] = zero16
        return 0

    lax.fori_loop(0, CHUNK, zero_zrows, 0)

    def zero_av(i, _):
        a_v[pl.ds(i * 16, 16)] = zero16
        return 0

    lax.fori_loop(0, CHUNK // 16, zero_av, 0)

    row0 = s * 640
    pltpu.sync_copy(zrows.at[pl.ds(0, 256)], acc_s.at[pl.ds(row0, 256)])
    pltpu.sync_copy(zrows.at[pl.ds(0, 256)], acc_s.at[pl.ds(row0 + 256, 256)])
    pltpu.sync_copy(zrows.at[pl.ds(0, 128)], acc_s.at[pl.ds(row0 + 512, 128)])
    pltpu.sync_copy(a_v.at[pl.ds(0, 256)], den_s.at[pl.ds(row0, 256)])
    pltpu.sync_copy(a_v.at[pl.ds(0, 256)], den_s.at[pl.ds(row0 + 256, 256)])
    pltpu.sync_copy(a_v.at[pl.ds(0, 128)], den_s.at[pl.ds(row0 + 512, 128)])

    pltpu.sync_copy(cv_hbm.at[pl.ds(c * 16, 16)], cv_l)

    plsc.subcore_barrier()

    cv = cv_l[...]

    def chunk_body(g, _):
        chunk_id = g * 16 + s
        row = chunk_id * NSUB
        h1 = pltpu.async_copy(srcl_hbm.at[pl.ds(row, NSUB)], src_v, sem)
        h2 = pltpu.async_copy(dstl_hbm.at[pl.ds(row, NSUB)], dst_v, sem)
        h1.wait()
        h2.wait()

        for j in range(NSUB):
            for i in range(8):
                off = i * 16
                sv = src_v[j, pl.ds(off, 16)]
                src_v[j, pl.ds(off, 16)] = sv + cN
                dv = dst_v[j, pl.ds(off, 16)]
                dstg_v[j, pl.ds(off, 16)] = jnp.minimum(dv, N - 1) + cN

        hs = []
        for j in range(NSUB):
            hs.append(pltpu.async_copy(esn_hbm.at[src_v.at[j]],
                                       es_c.at[pl.ds(j * 128, 128)], sem))
            hs.append(pltpu.async_copy(edn_hbm.at[dstg_v.at[j]],
                                       ed_c.at[pl.ds(j * 128, 128)], sem))
            hs.append(pltpu.async_copy(z_hbm.at[src_v.at[j]],
                                       zrows.at[pl.ds(j * 128, 128)], sem))
        for h in hs:
            h.wait()

        for j in range(NSUB):
            for i in range(8):
                off = j * 128 + i * 16
                e = es_c[pl.ds(off, 16)] + ed_c[pl.ds(off, 16)]
                e = jnp.where(e > 0, e, 0.2 * e)
                a_v[pl.ds(off, 16)] = jnp.exp(e - cv)

        def scale_rows16(t, _):
            a16 = a_v[pl.ds(t * 16, 16)]
            for j in range(16):
                r = t * 16 + j
                a_sc = a16[j]
                for k in range(8):
                    zrows[r, pl.ds(k * 16, 16)] = (
                        zrows[r, pl.ds(k * 16, 16)] * a_sc)
            return 0

        lax.fori_loop(0, CHUNK // 16, scale_rows16, 0)

        ss = []
        for j in range(NSUB):
            ss.append(pltpu.async_copy(a_v.at[pl.ds(j * 128, 128)],
                                       den_s.at[dst_v.at[j]], sem, add=True))
            ss.append(pltpu.async_copy(zrows.at[pl.ds(j * 128, 128)],
                                       acc_s.at[dst_v.at[j]], sem, add=True))
        for h in ss:
            h.wait()
        return 0

    lax.fori_loop(0, NCHUNK, chunk_body, 0)

    plsc.subcore_barrier()

    pltpu.sync_copy(acc_s.at[pl.ds(row0, 640)],
                    acc_hbm.at[pl.ds(c * NP + row0, 640)])
    pltpu.sync_copy(den_s.at[pl.ds(row0, 640)],
                    den_hbm.at[pl.ds(c * NP + row0, 640)])


def _sc_edge(z, esn, edn, srcl, dstl, cvec):
    mesh = plsc.VectorSubcoreMesh(core_axis_name="c", subcore_axis_name="s")
    fn = pl.kernel(
        _sc_edge_body,
        mesh=mesh,
        compiler_params=pltpu.CompilerParams(needs_layout_passes=False),
        out_type=[
            jax.ShapeDtypeStruct((HEADS * NP, H), jnp.float32),
            jax.ShapeDtypeStruct((HEADS * NP,), jnp.float32),
        ],
        scratch_types=[
            pltpu.VMEM((CHUNK, H), jnp.float32),
            pltpu.VMEM((NSUB, 128), jnp.int32),
            pltpu.VMEM((NSUB, 128), jnp.int32),
            pltpu.VMEM((NSUB, 128), jnp.int32),
            pltpu.VMEM((CHUNK,), jnp.float32),
            pltpu.VMEM((CHUNK,), jnp.float32),
            pltpu.VMEM((CHUNK,), jnp.float32),
            pltpu.VMEM((16,), jnp.float32),
            pltpu.VMEM_SHARED((NP, H), jnp.float32),
            pltpu.VMEM_SHARED((NP,), jnp.float32),
            pltpu.SemaphoreType.DMA,
        ],
    )
    return fn(z, esn, edn, srcl, dstl, cvec)


def _prep_scores(es, ed):
    es2 = es.reshape(HEADS, N)
    ed2 = ed.reshape(HEADS, N)
    c2 = jax.nn.relu(jnp.max(es2, axis=1) + jnp.max(ed2, axis=1))
    cvec = jnp.repeat(c2, 16)
    return es.reshape(-1), ed.reshape(-1), cvec


def _split_heads(acc, den):
    accA = acc[:N]
    accB = acc[NP:NP + N]
    denA = jnp.maximum(den[:N], 1e-38)[:, None]
    denB = jnp.maximum(den[NP:NP + N], 1e-38)[:, None]
    return accA, accB, denA, denB


def kernel(x, edge_index, W0, as0, ad0, W1, as1, ad1, gWih, gWhh, gbih, gbhh,
           s_w, out_W):
    src = edge_index[0]
    dst = edge_index[1]
    srcl = jnp.pad(src, (0, EPAD - E)).reshape(-1, 128)
    dstl = jnp.pad(dst, (0, EPAD - E), constant_values=N).reshape(-1, 128)

    z0, es0, ed0 = _proj(x, W0, as0, ad0)
    esn0, edn0, cv0 = _prep_scores(es0, ed0)
    acc0, den0 = _sc_edge(z0, esn0, edn0, srcl, dstl, cv0)
    a0A, a0B, d0A, d0B = _split_heads(acc0, den0)

    h1, z1, es1, ed1 = _norm_proj(a0A, a0B, d0A, d0B, W1, as1, ad1)
    esn1, edn1, cv1 = _prep_scores(es1, ed1)
    acc1, den1 = _sc_edge(z1, esn1, edn1, srcl, dstl, cv1)
    a1A, a1B, d1A, d1B = _split_heads(acc1, den1)

    return _gru_out(h1, a1A, a1B, d1A, d1B, gWih, gWhh, gbih, gbhh, out_W)


# speedup vs baseline: 30.4452x; 1.4875x over previous
import functools

import jax
import jax.numpy as jnp
from jax import lax
from jax.experimental import pallas as pl
from jax.experimental.pallas import tpu as pltpu
from jax.experimental.pallas import tpu_sc as plsc

N = 10000
E = 160000
IN = 256
H = 128
HEADS = 2
HID = 256
OUT = 256

NP = 10240
CHUNK = 96
NCHUNK = 105
EPAD = 16 * NCHUNK * CHUNK

_B = 2000


def _dotT(a, b):
    return lax.dot_general(a, b, (((1,), (1,)), ((), ())),
                           preferred_element_type=jnp.float32)


def _proj_body(hin_ref, W_ref, as_ref, ad_ref, z_ref, es_ref, ed_ref):
    h = pl.program_id(0)
    z = _dotT(hin_ref[...], W_ref[h])
    z_ref[...] = z
    es_ref[...] = (z @ as_ref[h])[:, None]
    ed_ref[...] = (z @ ad_ref[h])[:, None]


def _proj(hin, W, a_s, a_d):
    grid = (HEADS, N // _B)
    full = lambda shape: pl.BlockSpec(shape, lambda h, i: (0,) * len(shape))
    return pl.pallas_call(
        _proj_body,
        grid=grid,
        in_specs=[
            pl.BlockSpec((_B, IN), lambda h, i: (i, 0)),
            full((HEADS, H, IN)),
            full((HEADS, H)),
            full((HEADS, H)),
        ],
        out_specs=[
            pl.BlockSpec((_B, H), lambda h, i: (h * (N // _B) + i, 0)),
            pl.BlockSpec((_B, 1), lambda h, i: (h * (N // _B) + i, 0)),
            pl.BlockSpec((_B, 1), lambda h, i: (h * (N // _B) + i, 0)),
        ],
        out_shape=[
            jax.ShapeDtypeStruct((HEADS * N, H), jnp.float32),
            jax.ShapeDtypeStruct((HEADS * N, 1), jnp.float32),
            jax.ShapeDtypeStruct((HEADS * N, 1), jnp.float32),
        ],
    )(hin, W, a_s, a_d)


def _norm_proj_body(accA_ref, accB_ref, denA_ref, denB_ref, W_ref, as_ref,
                    ad_ref, hcat_ref, z_ref, es_ref, ed_ref):
    hA = jax.nn.relu(accA_ref[...] / denA_ref[...])
    hB = jax.nn.relu(accB_ref[...] / denB_ref[...])
    hcat = jnp.concatenate([hA, hB], axis=1)
    hcat_ref[...] = hcat
    h = pl.program_id(0)
    z = _dotT(hcat, W_ref[h])
    z_ref[...] = z
    es_ref[...] = (z @ as_ref[h])[:, None]
    ed_ref[...] = (z @ ad_ref[h])[:, None]


def _norm_proj(accA, accB, denA, denB, W, a_s, a_d):
    grid = (HEADS, N // _B)
    full = lambda shape: pl.BlockSpec(shape, lambda h, i: (0,) * len(shape))
    return pl.pallas_call(
        _norm_proj_body,
        grid=grid,
        in_specs=[
            pl.BlockSpec((_B, H), lambda h, i: (i, 0)),
            pl.BlockSpec((_B, H), lambda h, i: (i, 0)),
            pl.BlockSpec((_B, 1), lambda h, i: (i, 0)),
            pl.BlockSpec((_B, 1), lambda h, i: (i, 0)),
            full((HEADS, H, 2 * H)),
            full((HEADS, H)),
            full((HEADS, H)),
        ],
        out_specs=[
            pl.BlockSpec((_B, 2 * H), lambda h, i: (i, 0)),
            pl.BlockSpec((_B, H), lambda h, i: (h * (N // _B) + i, 0)),
            pl.BlockSpec((_B, 1), lambda h, i: (h * (N // _B) + i, 0)),
            pl.BlockSpec((_B, 1), lambda h, i: (h * (N // _B) + i, 0)),
        ],
        out_shape=[
            jax.ShapeDtypeStruct((N, 2 * H), jnp.float32),
            jax.ShapeDtypeStruct((HEADS * N, H), jnp.float32),
            jax.ShapeDtypeStruct((HEADS * N, 1), jnp.float32),
            jax.ShapeDtypeStruct((HEADS * N, 1), jnp.float32),
        ],
    )(accA, accB, denA, denB, W, a_s, a_d)


def _gru_out_body(h1_ref, accA_ref, accB_ref, denA_ref, denB_ref, gWih_ref,
                  gWhh_ref, gbih_ref, gbhh_ref, outW_ref, o_ref):
    h1 = h1_ref[...]
    hA = jax.nn.relu(accA_ref[...] / denA_ref[...])
    hB = jax.nn.relu(accB_ref[...] / denB_ref[...])
    h2 = jnp.concatenate([hA, hB], axis=1)

    def step(xs, h, Wih, Whh, bih, bhh, with_h):
        gi = _dotT(xs, Wih) + bih
        if with_h:
            gh = _dotT(h, Whh) + bhh
        else:
            gh = jnp.broadcast_to(bhh, gi.shape)
        i_r, i_z, i_n = gi[:, :HID], gi[:, HID:2 * HID], gi[:, 2 * HID:]
        h_r, h_z, h_n = gh[:, :HID], gh[:, HID:2 * HID], gh[:, 2 * HID:]
        r = jax.nn.sigmoid(i_r + h_r)
        zg = jax.nn.sigmoid(i_z + h_z)
        n = jnp.tanh(i_n + r * h_n)
        if with_h:
            return (1.0 - zg) * n + zg * h
        return (1.0 - zg) * n

    f0 = step(h1, None, gWih_ref[0], gWhh_ref[0], gbih_ref[0], gbhh_ref[0],
              False)
    f1 = step(h2, f0, gWih_ref[0], gWhh_ref[0], gbih_ref[0], gbhh_ref[0],
              True)
    b0 = step(h2, None, gWih_ref[1], gWhh_ref[1], gbih_ref[1], gbhh_ref[1],
              False)
    b1 = step(h1, b0, gWih_ref[1], gWhh_ref[1], gbih_ref[1], gbhh_ref[1],
              True)

    outW = outW_ref[...]
    o_ref[...] = (_dotT(f0 + f1, outW[:, :HID]) +
                  _dotT(b0 + b1, outW[:, HID:]))


def _gru_out(h1, accA, accB, denA, denB, gWih, gWhh, gbih, gbhh, out_W):
    grid = (N // _B,)
    full = lambda shape: pl.BlockSpec(shape, lambda i: (0,) * len(shape))
    return pl.pallas_call(
        _gru_out_body,
        grid=grid,
        in_specs=[
            pl.BlockSpec((_B, HID), lambda i: (i, 0)),
            pl.BlockSpec((_B, H), lambda i: (i, 0)),
            pl.BlockSpec((_B, H), lambda i: (i, 0)),
            pl.BlockSpec((_B, 1), lambda i: (i, 0)),
            pl.BlockSpec((_B, 1), lambda i: (i, 0)),
            full((2, 3 * HID, HID)),
            full((2, 3 * HID, HID)),
            full((2, 3 * HID)),
            full((2, 3 * HID)),
            full((OUT, 2 * HID)),
        ],
        out_specs=pl.BlockSpec((_B, OUT), lambda i: (i, 0)),
        out_shape=jax.ShapeDtypeStruct((N, OUT), jnp.float32),
    )(h1, accA, accB, denA, denB, gWih, gWhh, gbih, gbhh, out_W)


def _sc_edge_body(z_hbm, esn_hbm, edn_hbm, srcl_hbm, dstl_hbm, cv_hbm,
                  acc_hbm, den_hbm,
                  zrows0, src0, dst0, dstg0, a0, es0, ed0,
                  zrows1, src1, dst1, dstg1, a1, es1, ed1,
                  zrows2, src2, dst2, dstg2, a2, es2, ed2,
                  cv_l, acc_s, den_s,
                  sem_i, semg0, semg1, semg2, sems0, sems1, sems2):
    c = lax.axis_index("c")
    s = lax.axis_index("s")
    cN = c * N

    bufs = [
        (zrows0, src0, dst0, dstg0, a0, es0, ed0, semg0, sems0),
        (zrows1, src1, dst1, dstg1, a1, es1, ed1, semg1, sems1),
        (zrows2, src2, dst2, dstg2, a2, es2, ed2, semg2, sems2),
    ]

    zero16 = jnp.zeros((16,), jnp.float32)

    def zero_zrows(r, _):
        for k in range(8):
            zrows0[r, pl.ds(k * 16, 16)] = zero16
        return 0

    lax.fori_loop(0, CHUNK, zero_zrows, 0)
    for i in range(CHUNK // 16):
        a0[pl.ds(i * 16, 16)] = zero16

    row0 = s * 640
    for k in range(6):
        pltpu.sync_copy(zrows0.at[pl.ds(0, 96)],
                        acc_s.at[pl.ds(row0 + 96 * k, 96)])
        pltpu.sync_copy(a0.at[pl.ds(0, 96)],
                        den_s.at[pl.ds(row0 + 96 * k, 96)])
    pltpu.sync_copy(zrows0.at[pl.ds(0, 64)], acc_s.at[pl.ds(row0 + 576, 64)])
    pltpu.sync_copy(a0.at[pl.ds(0, 64)], den_s.at[pl.ds(row0 + 576, 64)])

    pltpu.sync_copy(cv_hbm.at[pl.ds(c * 16, 16)], cv_l)

    plsc.subcore_barrier()

    cv = cv_l[...]


    def prep(j, B):
        zr, sv, dv, dg, av, ec, dc, sg, ss = B
        row = j * 16 + s
        h1 = pltpu.async_copy(srcl_hbm.at[pl.ds(row, 1)], sv, sem_i)
        h2 = pltpu.async_copy(dstl_hbm.at[pl.ds(row, 1)], dv, sem_i)
        h1.wait()
        h2.wait()
        for i in range(CHUNK // 16):
            off = i * 16
            sv[0, pl.ds(off, 16)] = sv[0, pl.ds(off, 16)] + cN
            dvv = dv[0, pl.ds(off, 16)]
            dg[0, pl.ds(off, 16)] = jnp.minimum(dvv, N - 1) + cN
        pltpu.async_copy(esn_hbm.at[sv.at[0]], ec, sg)
        pltpu.async_copy(edn_hbm.at[dg.at[0]], dc, sg)
        pltpu.async_copy(z_hbm.at[sv.at[0]], zr, sg)

    def drain_scatters(B):
        zr, sv, dv, dg, av, ec, dc, sg, ss = B
        pltpu.make_async_copy(av, den_s.at[dv.at[0]], ss).wait()
        pltpu.make_async_copy(zr, acc_s.at[dv.at[0]], ss).wait()

    def compute(B):
        zr, sv, dv, dg, av, ec, dc, sg, ss = B
        pltpu.make_async_copy(esn_hbm.at[sv.at[0]], ec, sg).wait()
        pltpu.make_async_copy(edn_hbm.at[dg.at[0]], dc, sg).wait()
        pltpu.make_async_copy(z_hbm.at[sv.at[0]], zr, sg).wait()

        for i in range(CHUNK // 16):
            off = i * 16
            e = ec[pl.ds(off, 16)] + dc[pl.ds(off, 16)]
            e = jnp.where(e > 0, e, 0.2 * e)
            av[pl.ds(off, 16)] = jnp.exp(e - cv)

        def scale_rows16(t, _):
            a16 = av[pl.ds(t * 16, 16)]
            for jj in range(16):
                r = t * 16 + jj
                a_sc = a16[jj]
                for k in range(8):
                    zr[r, pl.ds(k * 16, 16)] = zr[r, pl.ds(k * 16, 16)] * a_sc
            return 0

        lax.fori_loop(0, CHUNK // 16, scale_rows16, 0)

        pltpu.async_copy(av, den_s.at[dv.at[0]], ss, add=True)
        pltpu.async_copy(zr, acc_s.at[dv.at[0]], ss, add=True)

    prep(0, bufs[0])
    prep(1, bufs[1])

    def group_body(i, _):
        for k in range(3):
            g = 3 * i + k
            j = g + 2
            B_next = bufs[(k + 2) % 3]

            @pl.when(j < NCHUNK)
            def _():
                @pl.when(j >= 3)
                def _():
                    drain_scatters(B_next)
                prep(j, B_next)

            compute(bufs[k])
        return 0

    lax.fori_loop(0, NCHUNK // 3, group_body, 0)

    for k in range(3):
        drain_scatters(bufs[k])

    plsc.subcore_barrier()

    pltpu.sync_copy(acc_s.at[pl.ds(row0, 640)],
                    acc_hbm.at[pl.ds(c * NP + row0, 640)])
    pltpu.sync_copy(den_s.at[pl.ds(row0, 640)],
                    den_hbm.at[pl.ds(c * NP + row0, 640)])


def _sc_edge(z, esn, edn, srcl, dstl, cvec):
    mesh = plsc.VectorSubcoreMesh(core_axis_name="c", subcore_axis_name="s")
    buf_set = [
        pltpu.VMEM((CHUNK, H), jnp.float32),
        pltpu.VMEM((1, CHUNK), jnp.int32),
        pltpu.VMEM((1, CHUNK), jnp.int32),
        pltpu.VMEM((1, CHUNK), jnp.int32),
        pltpu.VMEM((CHUNK,), jnp.float32),
        pltpu.VMEM((CHUNK,), jnp.float32),
        pltpu.VMEM((CHUNK,), jnp.float32),
    ]
    fn = pl.kernel(
        _sc_edge_body,
        mesh=mesh,
        compiler_params=pltpu.CompilerParams(needs_layout_passes=False),
        out_type=[
            jax.ShapeDtypeStruct((HEADS * NP, H), jnp.float32),
            jax.ShapeDtypeStruct((HEADS * NP,), jnp.float32),
        ],
        scratch_types=(
            buf_set * 3
            + [
                pltpu.VMEM((16,), jnp.float32),
                pltpu.VMEM_SHARED((NP, H), jnp.float32),
                pltpu.VMEM_SHARED((NP,), jnp.float32),
            ]
            + [pltpu.SemaphoreType.DMA] * 7
        ),
    )
    return fn(z, esn, edn, srcl, dstl, cvec)


def _prep_scores(es, ed):
    es2 = es.reshape(HEADS, N)
    ed2 = ed.reshape(HEADS, N)
    c2 = jax.nn.relu(jnp.max(es2, axis=1) + jnp.max(ed2, axis=1))
    cvec = jnp.repeat(c2, 16)
    return es.reshape(-1), ed.reshape(-1), cvec


def _split_heads(acc, den):
    accA = acc[:N]
    accB = acc[NP:NP + N]
    denA = jnp.maximum(den[:N], 1e-38)[:, None]
    denB = jnp.maximum(den[NP:NP + N], 1e-38)[:, None]
    return accA, accB, denA, denB


def kernel(x, edge_index, W0, as0, ad0, W1, as1, ad1, gWih, gWhh, gbih, gbhh,
           s_w, out_W):
    src = edge_index[0]
    dst = edge_index[1]
    srcl = jnp.pad(src, (0, EPAD - E)).reshape(-1, CHUNK)
    dstl = jnp.pad(dst, (0, EPAD - E), constant_values=N).reshape(-1, CHUNK)

    z0, es0, ed0 = _proj(x, W0, as0, ad0)
    esn0, edn0, cv0 = _prep_scores(es0, ed0)
    acc0, den0 = _sc_edge(z0, esn0, edn0, srcl, dstl, cv0)
    a0A, a0B, d0A, d0B = _split_heads(acc0, den0)

    h1, z1, es1, ed1 = _norm_proj(a0A, a0B, d0A, d0B, W1, as1, ad1)
    esn1, edn1, cv1 = _prep_scores(es1, ed1)
    acc1, den1 = _sc_edge(z1, esn1, edn1, srcl, dstl, cv1)
    a1A, a1B, d1A, d1B = _split_heads(acc1, den1)

    return _gru_out(h1, a1A, a1B, d1A, d1B, gWih, gWhh, gbih, gbhh, out_W)


# speedup vs baseline: 34.8646x; 1.1452x over previous
import functools

import jax
import jax.numpy as jnp
from jax import lax
from jax.experimental import pallas as pl
from jax.experimental.pallas import tpu as pltpu
from jax.experimental.pallas import tpu_sc as plsc

N = 10000
E = 160000
IN = 256
H = 128
HEADS = 2
HID = 256
OUT = 256

NP = 10240
CHUNK = 96
NCHUNK = 105
EPAD = 16 * NCHUNK * CHUNK

_B = 2000


def _dotT(a, b):
    return lax.dot_general(a, b, (((1,), (1,)), ((), ())),
                           preferred_element_type=jnp.float32)


def _proj_body(hin_ref, W_ref, as_ref, ad_ref, z_ref, es_ref, ed_ref):
    h = pl.program_id(0)
    z = _dotT(hin_ref[...], W_ref[h])
    z_ref[...] = z
    es_ref[...] = (z @ as_ref[h])[:, None]
    ed_ref[...] = (z @ ad_ref[h])[:, None]


def _proj(hin, W, a_s, a_d):
    grid = (HEADS, N // _B)
    full = lambda shape: pl.BlockSpec(shape, lambda h, i: (0,) * len(shape))
    return pl.pallas_call(
        _proj_body,
        grid=grid,
        in_specs=[
            pl.BlockSpec((_B, IN), lambda h, i: (i, 0)),
            full((HEADS, H, IN)),
            full((HEADS, H)),
            full((HEADS, H)),
        ],
        out_specs=[
            pl.BlockSpec((_B, H), lambda h, i: (h * (N // _B) + i, 0)),
            pl.BlockSpec((_B, 1), lambda h, i: (h * (N // _B) + i, 0)),
            pl.BlockSpec((_B, 1), lambda h, i: (h * (N // _B) + i, 0)),
        ],
        out_shape=[
            jax.ShapeDtypeStruct((HEADS * N, H), jnp.float32),
            jax.ShapeDtypeStruct((HEADS * N, 1), jnp.float32),
            jax.ShapeDtypeStruct((HEADS * N, 1), jnp.float32),
        ],
    )(hin, W, a_s, a_d)


def _norm_proj_body(accA_ref, accB_ref, denA_ref, denB_ref, W_ref, as_ref,
                    ad_ref, hcat_ref, z_ref, es_ref, ed_ref):
    hA = jax.nn.relu(accA_ref[...] / denA_ref[...])
    hB = jax.nn.relu(accB_ref[...] / denB_ref[...])
    hcat = jnp.concatenate([hA, hB], axis=1)
    hcat_ref[...] = hcat
    h = pl.program_id(0)
    z = _dotT(hcat, W_ref[h])
    z_ref[...] = z
    es_ref[...] = (z @ as_ref[h])[:, None]
    ed_ref[...] = (z @ ad_ref[h])[:, None]


def _norm_proj(accA, accB, denA, denB, W, a_s, a_d):
    grid = (HEADS, N // _B)
    full = lambda shape: pl.BlockSpec(shape, lambda h, i: (0,) * len(shape))
    return pl.pallas_call(
        _norm_proj_body,
        grid=grid,
        in_specs=[
            pl.BlockSpec((_B, H), lambda h, i: (i, 0)),
            pl.BlockSpec((_B, H), lambda h, i: (i, 0)),
            pl.BlockSpec((_B, 1), lambda h, i: (i, 0)),
            pl.BlockSpec((_B, 1), lambda h, i: (i, 0)),
            full((HEADS, H, 2 * H)),
            full((HEADS, H)),
            full((HEADS, H)),
        ],
        out_specs=[
            pl.BlockSpec((_B, 2 * H), lambda h, i: (i, 0)),
            pl.BlockSpec((_B, H), lambda h, i: (h * (N // _B) + i, 0)),
            pl.BlockSpec((_B, 1), lambda h, i: (h * (N // _B) + i, 0)),
            pl.BlockSpec((_B, 1), lambda h, i: (h * (N // _B) + i, 0)),
        ],
        out_shape=[
            jax.ShapeDtypeStruct((N, 2 * H), jnp.float32),
            jax.ShapeDtypeStruct((HEADS * N, H), jnp.float32),
            jax.ShapeDtypeStruct((HEADS * N, 1), jnp.float32),
            jax.ShapeDtypeStruct((HEADS * N, 1), jnp.float32),
        ],
    )(accA, accB, denA, denB, W, a_s, a_d)


def _gru_out_body(h1_ref, accA_ref, accB_ref, denA_ref, denB_ref, gWih_ref,
                  gWhh_ref, gbih_ref, gbhh_ref, outW_ref, o_ref):
    h1 = h1_ref[...]
    hA = jax.nn.relu(accA_ref[...] / denA_ref[...])
    hB = jax.nn.relu(accB_ref[...] / denB_ref[...])
    h2 = jnp.concatenate([hA, hB], axis=1)

    def step(xs, h, Wih, Whh, bih, bhh, with_h):
        gi = _dotT(xs, Wih) + bih
        if with_h:
            gh = _dotT(h, Whh) + bhh
        else:
            gh = jnp.broadcast_to(bhh, gi.shape)
        i_r, i_z, i_n = gi[:, :HID], gi[:, HID:2 * HID], gi[:, 2 * HID:]
        h_r, h_z, h_n = gh[:, :HID], gh[:, HID:2 * HID], gh[:, 2 * HID:]
        r = jax.nn.sigmoid(i_r + h_r)
        zg = jax.nn.sigmoid(i_z + h_z)
        n = jnp.tanh(i_n + r * h_n)
        if with_h:
            return (1.0 - zg) * n + zg * h
        return (1.0 - zg) * n

    f0 = step(h1, None, gWih_ref[0], gWhh_ref[0], gbih_ref[0], gbhh_ref[0],
              False)
    f1 = step(h2, f0, gWih_ref[0], gWhh_ref[0], gbih_ref[0], gbhh_ref[0],
              True)
    b0 = step(h2, None, gWih_ref[1], gWhh_ref[1], gbih_ref[1], gbhh_ref[1],
              False)
    b1 = step(h1, b0, gWih_ref[1], gWhh_ref[1], gbih_ref[1], gbhh_ref[1],
              True)

    outW = outW_ref[...]
    o_ref[...] = (_dotT(f0 + f1, outW[:, :HID]) +
                  _dotT(b0 + b1, outW[:, HID:]))


def _gru_out(h1, accA, accB, denA, denB, gWih, gWhh, gbih, gbhh, out_W):
    grid = (N // _B,)
    full = lambda shape: pl.BlockSpec(shape, lambda i: (0,) * len(shape))
    return pl.pallas_call(
        _gru_out_body,
        grid=grid,
        in_specs=[
            pl.BlockSpec((_B, HID), lambda i: (i, 0)),
            pl.BlockSpec((_B, H), lambda i: (i, 0)),
            pl.BlockSpec((_B, H), lambda i: (i, 0)),
            pl.BlockSpec((_B, 1), lambda i: (i, 0)),
            pl.BlockSpec((_B, 1), lambda i: (i, 0)),
            full((2, 3 * HID, HID)),
            full((2, 3 * HID, HID)),
            full((2, 3 * HID)),
            full((2, 3 * HID)),
            full((OUT, 2 * HID)),
        ],
        out_specs=pl.BlockSpec((_B, OUT), lambda i: (i, 0)),
        out_shape=jax.ShapeDtypeStruct((N, OUT), jnp.float32),
    )(h1, accA, accB, denA, denB, gWih, gWhh, gbih, gbhh, out_W)


def _sc_edge_body(z_hbm, esn_hbm, edn_hbm, srcl_hbm, dstl_hbm, cv_hbm,
                  acc_hbm, den_hbm,
                  zrows0, src0, dst0, dstg0, a0, es0, ed0,
                  zrows1, src1, dst1, dstg1, a1, es1, ed1,
                  zrows2, src2, dst2, dstg2, a2, es2, ed2,
                  pss0, psd0, pss1, psd1, pss2, psd2,
                  cv_l, acc_s, den_s,
                  sem_i0, sem_i1, sem_i2,
                  semg0, semg1, semg2, sems0, sems1, sems2):
    c = lax.axis_index("c")
    s = lax.axis_index("s")
    cN = c * N

    bufs = [
        (zrows0, src0, dst0, dstg0, a0, es0, ed0, semg0, sems0),
        (zrows1, src1, dst1, dstg1, a1, es1, ed1, semg1, sems1),
        (zrows2, src2, dst2, dstg2, a2, es2, ed2, semg2, sems2),
    ]
    stages = [(pss0, psd0, sem_i0), (pss1, psd1, sem_i1), (pss2, psd2, sem_i2)]

    zero16 = jnp.zeros((16,), jnp.float32)

    def zero_zrows(r, _):
        for k in range(8):
            zrows0[r, pl.ds(k * 16, 16)] = zero16
        return 0

    lax.fori_loop(0, CHUNK, zero_zrows, 0)
    for i in range(CHUNK // 16):
        a0[pl.ds(i * 16, 16)] = zero16

    row0 = s * 640
    for k in range(6):
        pltpu.sync_copy(zrows0.at[pl.ds(0, 96)],
                        acc_s.at[pl.ds(row0 + 96 * k, 96)])
        pltpu.sync_copy(a0.at[pl.ds(0, 96)],
                        den_s.at[pl.ds(row0 + 96 * k, 96)])
    pltpu.sync_copy(zrows0.at[pl.ds(0, 64)], acc_s.at[pl.ds(row0 + 576, 64)])
    pltpu.sync_copy(a0.at[pl.ds(0, 64)], den_s.at[pl.ds(row0 + 576, 64)])

    pltpu.sync_copy(cv_hbm.at[pl.ds(c * 16, 16)], cv_l)

    plsc.subcore_barrier()

    cv = cv_l[...]


    def fire_idx(j, stage):
        ssrc, sdst, sip = stage
        row = j * 16 + s
        pltpu.async_copy(srcl_hbm.at[pl.ds(row, 1)], ssrc, sip)
        pltpu.async_copy(dstl_hbm.at[pl.ds(row, 1)], sdst, sip)

    def prep(j, B, stage, in_loop):
        zr, sv, dv, dg, av, ec, dc, sg, ss = B
        ssrc, sdst, sip = stage
        row = j * 16 + s
        pltpu.make_async_copy(srcl_hbm.at[pl.ds(row, 1)], ssrc, sip).wait()
        pltpu.make_async_copy(dstl_hbm.at[pl.ds(row, 1)], sdst, sip).wait()
        for i in range(CHUNK // 16):
            off = i * 16
            sv[0, pl.ds(off, 16)] = ssrc[0, pl.ds(off, 16)] + cN
            dvv = sdst[0, pl.ds(off, 16)]
            dv[0, pl.ds(off, 16)] = dvv
            dg[0, pl.ds(off, 16)] = jnp.minimum(dvv, N - 1) + cN

        if in_loop:
            @pl.when(j + 3 < NCHUNK)
            def _():
                fire_idx(j + 3, stage)
        else:
            fire_idx(j + 3, stage)

        pltpu.async_copy(esn_hbm.at[sv.at[0]], ec, sg)
        pltpu.async_copy(edn_hbm.at[dg.at[0]], dc, sg)
        pltpu.async_copy(z_hbm.at[sv.at[0]], zr, sg)

    def drain_scatters(B):
        zr, sv, dv, dg, av, ec, dc, sg, ss = B
        pltpu.make_async_copy(av, den_s.at[dv.at[0]], ss).wait()
        pltpu.make_async_copy(zr, acc_s.at[dv.at[0]], ss).wait()

    def compute(B):
        zr, sv, dv, dg, av, ec, dc, sg, ss = B
        pltpu.make_async_copy(esn_hbm.at[sv.at[0]], ec, sg).wait()
        pltpu.make_async_copy(edn_hbm.at[dg.at[0]], dc, sg).wait()
        pltpu.make_async_copy(z_hbm.at[sv.at[0]], zr, sg).wait()

        for i in range(CHUNK // 16):
            off = i * 16
            e = ec[pl.ds(off, 16)] + dc[pl.ds(off, 16)]
            e = jnp.where(e > 0, e, 0.2 * e)
            av[pl.ds(off, 16)] = jnp.exp(e - cv)

        def scale_rows16(t, _):
            a16 = av[pl.ds(t * 16, 16)]
            for jj in range(16):
                r = t * 16 + jj
                a_sc = a16[jj]
                for k in range(8):
                    zr[r, pl.ds(k * 16, 16)] = zr[r, pl.ds(k * 16, 16)] * a_sc
            return 0

        lax.fori_loop(0, CHUNK // 16, scale_rows16, 0)

        pltpu.async_copy(av, den_s.at[dv.at[0]], ss, add=True)
        pltpu.async_copy(zr, acc_s.at[dv.at[0]], ss, add=True)

    fire_idx(0, stages[0])
    fire_idx(1, stages[1])
    fire_idx(2, stages[2])
    prep(0, bufs[0], stages[0], False)
    prep(1, bufs[1], stages[1], False)

    def group_body(i, _):
        for k in range(3):
            g = 3 * i + k
            j = g + 2
            q = (k + 2) % 3
            B_next = bufs[q]

            @pl.when(j < NCHUNK)
            def _():
                @pl.when(j >= 3)
                def _():
                    drain_scatters(B_next)
                prep(j, B_next, stages[q], True)

            compute(bufs[k])
        return 0

    lax.fori_loop(0, NCHUNK // 3, group_body, 0)

    for k in range(3):
        drain_scatters(bufs[k])

    plsc.subcore_barrier()

    pltpu.sync_copy(acc_s.at[pl.ds(row0, 640)],
                    acc_hbm.at[pl.ds(c * NP + row0, 640)])
    pltpu.sync_copy(den_s.at[pl.ds(row0, 640)],
                    den_hbm.at[pl.ds(c * NP + row0, 640)])


def _sc_edge(z, esn, edn, srcl, dstl, cvec):
    mesh = plsc.VectorSubcoreMesh(core_axis_name="c", subcore_axis_name="s")
    buf_set = [
        pltpu.VMEM((CHUNK, H), jnp.float32),
        pltpu.VMEM((1, CHUNK), jnp.int32),
        pltpu.VMEM((1, CHUNK), jnp.int32),
        pltpu.VMEM((1, CHUNK), jnp.int32),
        pltpu.VMEM((CHUNK,), jnp.float32),
        pltpu.VMEM((CHUNK,), jnp.float32),
        pltpu.VMEM((CHUNK,), jnp.float32),
    ]
    fn = pl.kernel(
        _sc_edge_body,
        mesh=mesh,
        compiler_params=pltpu.CompilerParams(needs_layout_passes=False),
        out_type=[
            jax.ShapeDtypeStruct((HEADS * NP, H), jnp.float32),
            jax.ShapeDtypeStruct((HEADS * NP,), jnp.float32),
        ],
        scratch_types=(
            buf_set * 3
            + [pltpu.VMEM((1, CHUNK), jnp.int32)] * 6
            + [
                pltpu.VMEM((16,), jnp.float32),
                pltpu.VMEM_SHARED((NP, H), jnp.float32),
                pltpu.VMEM_SHARED((NP,), jnp.float32),
            ]
            + [pltpu.SemaphoreType.DMA] * 9
        ),
    )
    return fn(z, esn, edn, srcl, dstl, cvec)


def _prep_scores(es, ed):
    es2 = es.reshape(HEADS, N)
    ed2 = ed.reshape(HEADS, N)
    c2 = jax.nn.relu(jnp.max(es2, axis=1) + jnp.max(ed2, axis=1))
    cvec = jnp.repeat(c2, 16)
    return es.reshape(-1), ed.reshape(-1), cvec


def _split_heads(acc, den):
    accA = acc[:N]
    accB = acc[NP:NP + N]
    denA = jnp.maximum(den[:N], 1e-38)[:, None]
    denB = jnp.maximum(den[NP:NP + N], 1e-38)[:, None]
    return accA, accB, denA, denB


def kernel(x, edge_index, W0, as0, ad0, W1, as1, ad1, gWih, gWhh, gbih, gbhh,
           s_w, out_W):
    src = edge_index[0]
    dst = edge_index[1]
    srcl = jnp.pad(src, (0, EPAD - E)).reshape(-1, CHUNK)
    dstl = jnp.pad(dst, (0, EPAD - E), constant_values=N).reshape(-1, CHUNK)

    z0, es0, ed0 = _proj(x, W0, as0, ad0)
    esn0, edn0, cv0 = _prep_scores(es0, ed0)
    acc0, den0 = _sc_edge(z0, esn0, edn0, srcl, dstl, cv0)
    a0A, a0B, d0A, d0B = _split_heads(acc0, den0)

    h1, z1, es1, ed1 = _norm_proj(a0A, a0B, d0A, d0B, W1, as1, ad1)
    esn1, edn1, cv1 = _prep_scores(es1, ed1)
    acc1, den1 = _sc_edge(z1, esn1, edn1, srcl, dstl, cv1)
    a1A, a1B, d1A, d1B = _split_heads(acc1, den1)

    return _gru_out(h1, a1A, a1B, d1A, d1B, gWih, gWhh, gbih, gbhh, out_W)


# speedup vs baseline: 35.6074x; 1.0213x over previous
import functools

import jax
import jax.numpy as jnp
from jax import lax
from jax.experimental import pallas as pl
from jax.experimental.pallas import tpu as pltpu
from jax.experimental.pallas import tpu_sc as plsc

N = 10000
E = 160000
IN = 256
H = 128
HEADS = 2
HID = 256
OUT = 256

NP = 10240
CHUNK = 112
NCHUNK = 90
EPAD = 16 * NCHUNK * CHUNK

_B = 2000


def _dotT(a, b):
    return lax.dot_general(a, b, (((1,), (1,)), ((), ())),
                           preferred_element_type=jnp.float32)


def _proj_body(hin_ref, W_ref, as_ref, ad_ref, z_ref, es_ref, ed_ref):
    h = pl.program_id(0)
    z = _dotT(hin_ref[...], W_ref[h])
    z_ref[...] = z
    es_ref[...] = (z @ as_ref[h])[:, None]
    ed_ref[...] = (z @ ad_ref[h])[:, None]


def _proj(hin, W, a_s, a_d):
    grid = (HEADS, N // _B)
    full = lambda shape: pl.BlockSpec(shape, lambda h, i: (0,) * len(shape))
    return pl.pallas_call(
        _proj_body,
        grid=grid,
        in_specs=[
            pl.BlockSpec((_B, IN), lambda h, i: (i, 0)),
            full((HEADS, H, IN)),
            full((HEADS, H)),
            full((HEADS, H)),
        ],
        out_specs=[
            pl.BlockSpec((_B, H), lambda h, i: (h * (N // _B) + i, 0)),
            pl.BlockSpec((_B, 1), lambda h, i: (h * (N // _B) + i, 0)),
            pl.BlockSpec((_B, 1), lambda h, i: (h * (N // _B) + i, 0)),
        ],
        out_shape=[
            jax.ShapeDtypeStruct((HEADS * N, H), jnp.float32),
            jax.ShapeDtypeStruct((HEADS * N, 1), jnp.float32),
            jax.ShapeDtypeStruct((HEADS * N, 1), jnp.float32),
        ],
    )(hin, W, a_s, a_d)


def _norm_proj_body(accA_ref, accB_ref, denA_ref, denB_ref, W_ref, as_ref,
                    ad_ref, hcat_ref, z_ref, es_ref, ed_ref):
    hA = jax.nn.relu(accA_ref[...] / denA_ref[...])
    hB = jax.nn.relu(accB_ref[...] / denB_ref[...])
    hcat = jnp.concatenate([hA, hB], axis=1)
    hcat_ref[...] = hcat
    h = pl.program_id(0)
    z = _dotT(hcat, W_ref[h])
    z_ref[...] = z
    es_ref[...] = (z @ as_ref[h])[:, None]
    ed_ref[...] = (z @ ad_ref[h])[:, None]


def _norm_proj(accA, accB, denA, denB, W, a_s, a_d):
    grid = (HEADS, N // _B)
    full = lambda shape: pl.BlockSpec(shape, lambda h, i: (0,) * len(shape))
    return pl.pallas_call(
        _norm_proj_body,
        grid=grid,
        in_specs=[
            pl.BlockSpec((_B, H), lambda h, i: (i, 0)),
            pl.BlockSpec((_B, H), lambda h, i: (i, 0)),
            pl.BlockSpec((_B, 1), lambda h, i: (i, 0)),
            pl.BlockSpec((_B, 1), lambda h, i: (i, 0)),
            full((HEADS, H, 2 * H)),
            full((HEADS, H)),
            full((HEADS, H)),
        ],
        out_specs=[
            pl.BlockSpec((_B, 2 * H), lambda h, i: (i, 0)),
            pl.BlockSpec((_B, H), lambda h, i: (h * (N // _B) + i, 0)),
            pl.BlockSpec((_B, 1), lambda h, i: (h * (N // _B) + i, 0)),
            pl.BlockSpec((_B, 1), lambda h, i: (h * (N // _B) + i, 0)),
        ],
        out_shape=[
            jax.ShapeDtypeStruct((N, 2 * H), jnp.float32),
            jax.ShapeDtypeStruct((HEADS * N, H), jnp.float32),
            jax.ShapeDtypeStruct((HEADS * N, 1), jnp.float32),
            jax.ShapeDtypeStruct((HEADS * N, 1), jnp.float32),
        ],
    )(accA, accB, denA, denB, W, a_s, a_d)


def _gru_out_body(h1_ref, accA_ref, accB_ref, denA_ref, denB_ref, gWih_ref,
                  gWhh_ref, gbih_ref, gbhh_ref, outW_ref, o_ref):
    h1 = h1_ref[...]
    hA = jax.nn.relu(accA_ref[...] / denA_ref[...])
    hB = jax.nn.relu(accB_ref[...] / denB_ref[...])
    h2 = jnp.concatenate([hA, hB], axis=1)

    def step(xs, h, Wih, Whh, bih, bhh, with_h):
        gi = _dotT(xs, Wih) + bih
        if with_h:
            gh = _dotT(h, Whh) + bhh
        else:
            gh = jnp.broadcast_to(bhh, gi.shape)
        i_r, i_z, i_n = gi[:, :HID], gi[:, HID:2 * HID], gi[:, 2 * HID:]
        h_r, h_z, h_n = gh[:, :HID], gh[:, HID:2 * HID], gh[:, 2 * HID:]
        r = jax.nn.sigmoid(i_r + h_r)
        zg = jax.nn.sigmoid(i_z + h_z)
        n = jnp.tanh(i_n + r * h_n)
        if with_h:
            return (1.0 - zg) * n + zg * h
        return (1.0 - zg) * n

    f0 = step(h1, None, gWih_ref[0], gWhh_ref[0], gbih_ref[0], gbhh_ref[0],
              False)
    f1 = step(h2, f0, gWih_ref[0], gWhh_ref[0], gbih_ref[0], gbhh_ref[0],
              True)
    b0 = step(h2, None, gWih_ref[1], gWhh_ref[1], gbih_ref[1], gbhh_ref[1],
              False)
    b1 = step(h1, b0, gWih_ref[1], gWhh_ref[1], gbih_ref[1], gbhh_ref[1],
              True)

    outW = outW_ref[...]
    o_ref[...] = (_dotT(f0 + f1, outW[:, :HID]) +
                  _dotT(b0 + b1, outW[:, HID:]))


def _gru_out(h1, accA, accB, denA, denB, gWih, gWhh, gbih, gbhh, out_W):
    grid = (N // _B,)
    full = lambda shape: pl.BlockSpec(shape, lambda i: (0,) * len(shape))
    return pl.pallas_call(
        _gru_out_body,
        grid=grid,
        in_specs=[
            pl.BlockSpec((_B, HID), lambda i: (i, 0)),
            pl.BlockSpec((_B, H), lambda i: (i, 0)),
            pl.BlockSpec((_B, H), lambda i: (i, 0)),
            pl.BlockSpec((_B, 1), lambda i: (i, 0)),
            pl.BlockSpec((_B, 1), lambda i: (i, 0)),
            full((2, 3 * HID, HID)),
            full((2, 3 * HID, HID)),
            full((2, 3 * HID)),
            full((2, 3 * HID)),
            full((OUT, 2 * HID)),
        ],
        out_specs=pl.BlockSpec((_B, OUT), lambda i: (i, 0)),
        out_shape=jax.ShapeDtypeStruct((N, OUT), jnp.float32),
    )(h1, accA, accB, denA, denB, gWih, gWhh, gbih, gbhh, out_W)


def _sc_edge_body(z_hbm, esn_hbm, edn_hbm, srcl_hbm, dstl_hbm, cv_hbm,
                  acc_hbm, den_hbm,
                  zrows0, src0, dst0, dstg0, a0, es0, ed0,
                  zrows1, src1, dst1, dstg1, a1, es1, ed1,
                  zrows2, src2, dst2, dstg2, a2, es2, ed2,
                  pss0, psd0, pss1, psd1, pss2, psd2,
                  cv_l, acc_s, den_s,
                  sem_i0, sem_i1, sem_i2,
                  semg0, semg1, semg2, sems0, sems1, sems2):
    c = lax.axis_index("c")
    s = lax.axis_index("s")
    cN = c * N

    bufs = [
        (zrows0, src0, dst0, dstg0, a0, es0, ed0, semg0, sems0),
        (zrows1, src1, dst1, dstg1, a1, es1, ed1, semg1, sems1),
        (zrows2, src2, dst2, dstg2, a2, es2, ed2, semg2, sems2),
    ]
    stages = [(pss0, psd0, sem_i0), (pss1, psd1, sem_i1), (pss2, psd2, sem_i2)]

    zero16 = jnp.zeros((16,), jnp.float32)

    def zero_zrows(r, _):
        for k in range(8):
            zrows0[r, pl.ds(k * 16, 16)] = zero16
        return 0

    lax.fori_loop(0, CHUNK, zero_zrows, 0)
    for i in range(CHUNK // 16):
        a0[pl.ds(i * 16, 16)] = zero16

    row0 = s * 640
    for k in range(5):
        pltpu.sync_copy(zrows0.at[pl.ds(0, 112)],
                        acc_s.at[pl.ds(row0 + 112 * k, 112)])
        pltpu.sync_copy(a0.at[pl.ds(0, 112)],
                        den_s.at[pl.ds(row0 + 112 * k, 112)])
    pltpu.sync_copy(zrows0.at[pl.ds(0, 80)], acc_s.at[pl.ds(row0 + 560, 80)])
    pltpu.sync_copy(a0.at[pl.ds(0, 80)], den_s.at[pl.ds(row0 + 560, 80)])

    pltpu.sync_copy(cv_hbm.at[pl.ds(c * 16, 16)], cv_l)

    plsc.subcore_barrier()

    cv = cv_l[...]


    def fire_idx(j, stage):
        ssrc, sdst, sip = stage
        row = j * 16 + s
        pltpu.async_copy(srcl_hbm.at[pl.ds(row, 1)], ssrc, sip)
        pltpu.async_copy(dstl_hbm.at[pl.ds(row, 1)], sdst, sip)

    def prep(j, B, stage, in_loop):
        zr, sv, dv, dg, av, ec, dc, sg, ss = B
        ssrc, sdst, sip = stage
        row = j * 16 + s
        pltpu.make_async_copy(srcl_hbm.at[pl.ds(row, 1)], ssrc, sip).wait()
        pltpu.make_async_copy(dstl_hbm.at[pl.ds(row, 1)], sdst, sip).wait()
        for i in range(CHUNK // 16):
            off = i * 16
            sv[0, pl.ds(off, 16)] = ssrc[0, pl.ds(off, 16)] + cN
            dvv = sdst[0, pl.ds(off, 16)]
            dv[0, pl.ds(off, 16)] = dvv
            dg[0, pl.ds(off, 16)] = jnp.minimum(dvv, N - 1) + cN

        if in_loop:
            @pl.when(j + 3 < NCHUNK)
            def _():
                fire_idx(j + 3, stage)
        else:
            fire_idx(j + 3, stage)

        pltpu.async_copy(esn_hbm.at[sv.at[0]], ec, sg)
        pltpu.async_copy(edn_hbm.at[dg.at[0]], dc, sg)
        pltpu.async_copy(z_hbm.at[sv.at[0]], zr, sg)

    def drain_scatters(B):
        zr, sv, dv, dg, av, ec, dc, sg, ss = B
        pltpu.make_async_copy(av, den_s.at[dv.at[0]], ss).wait()
        pltpu.make_async_copy(zr, acc_s.at[dv.at[0]], ss).wait()

    def compute(B):
        zr, sv, dv, dg, av, ec, dc, sg, ss = B
        pltpu.make_async_copy(esn_hbm.at[sv.at[0]], ec, sg).wait()
        pltpu.make_async_copy(edn_hbm.at[dg.at[0]], dc, sg).wait()
        pltpu.make_async_copy(z_hbm.at[sv.at[0]], zr, sg).wait()

        for i in range(CHUNK // 16):
            off = i * 16
            e = ec[pl.ds(off, 16)] + dc[pl.ds(off, 16)]
            e = jnp.where(e > 0, e, 0.2 * e)
            av[pl.ds(off, 16)] = jnp.exp(e - cv)

        def scale_rows16(t, _):
            a16 = av[pl.ds(t * 16, 16)]
            for jj in range(16):
                r = t * 16 + jj
                a_sc = a16[jj]
                for k in range(8):
                    zr[r, pl.ds(k * 16, 16)] = zr[r, pl.ds(k * 16, 16)] * a_sc
            return 0

        lax.fori_loop(0, CHUNK // 16, scale_rows16, 0)

        pltpu.async_copy(av, den_s.at[dv.at[0]], ss, add=True)
        pltpu.async_copy(zr, acc_s.at[dv.at[0]], ss, add=True)

    fire_idx(0, stages[0])
    fire_idx(1, stages[1])
    fire_idx(2, stages[2])
    prep(0, bufs[0], stages[0], False)
    prep(1, bufs[1], stages[1], False)

    def group_body(i, _):
        for k in range(3):
            g = 3 * i + k
            j = g + 2
            q = (k + 2) % 3
            B_next = bufs[q]

            @pl.when(j < NCHUNK)
            def _():
                @pl.when(j >= 3)
                def _():
                    drain_scatters(B_next)
                prep(j, B_next, stages[q], True)

            compute(bufs[k])
        return 0

    lax.fori_loop(0, NCHUNK // 3, group_body, 0)

    for k in range(3):
        drain_scatters(bufs[k])

    plsc.subcore_barrier()

    pltpu.sync_copy(acc_s.at[pl.ds(row0, 640)],
                    acc_hbm.at[pl.ds(c * NP + row0, 640)])
    pltpu.sync_copy(den_s.at[pl.ds(row0, 640)],
                    den_hbm.at[pl.ds(c * NP + row0, 640)])


def _sc_edge(z, esn, edn, srcl, dstl, cvec):
    mesh = plsc.VectorSubcoreMesh(core_axis_name="c", subcore_axis_name="s")
    buf_set = [
        pltpu.VMEM((CHUNK, H), jnp.float32),
        pltpu.VMEM((1, CHUNK), jnp.int32),
        pltpu.VMEM((1, CHUNK), jnp.int32),
        pltpu.VMEM((1, CHUNK), jnp.int32),
        pltpu.VMEM((CHUNK,), jnp.float32),
        pltpu.VMEM((CHUNK,), jnp.float32),
        pltpu.VMEM((CHUNK,), jnp.float32),
    ]
    fn = pl.kernel(
        _sc_edge_body,
        mesh=mesh,
        compiler_params=pltpu.CompilerParams(needs_layout_passes=False),
        out_type=[
            jax.ShapeDtypeStruct((HEADS * NP, H), jnp.float32),
            jax.ShapeDtypeStruct((HEADS * NP,), jnp.float32),
        ],
        scratch_types=(
            buf_set * 3
            + [pltpu.VMEM((1, CHUNK), jnp.int32)] * 6
            + [
                pltpu.VMEM((16,), jnp.float32),
                pltpu.VMEM_SHARED((NP, H), jnp.float32),
                pltpu.VMEM_SHARED((NP,), jnp.float32),
            ]
            + [pltpu.SemaphoreType.DMA] * 9
        ),
    )
    return fn(z, esn, edn, srcl, dstl, cvec)


def _prep_scores(es, ed):
    es2 = es.reshape(HEADS, N)
    ed2 = ed.reshape(HEADS, N)
    c2 = jax.nn.relu(jnp.max(es2, axis=1) + jnp.max(ed2, axis=1))
    cvec = jnp.repeat(c2, 16)
    return es.reshape(-1), ed.reshape(-1), cvec


def _split_heads(acc, den):
    accA = acc[:N]
    accB = acc[NP:NP + N]
    denA = jnp.maximum(den[:N], 1e-38)[:, None]
    denB = jnp.maximum(den[NP:NP + N], 1e-38)[:, None]
    return accA, accB, denA, denB


def kernel(x, edge_index, W0, as0, ad0, W1, as1, ad1, gWih, gWhh, gbih, gbhh,
           s_w, out_W):
    src = edge_index[0]
    dst = edge_index[1]
    srcl = jnp.pad(src, (0, EPAD - E)).reshape(-1, CHUNK)
    dstl = jnp.pad(dst, (0, EPAD - E), constant_values=N).reshape(-1, CHUNK)

    z0, es0, ed0 = _proj(x, W0, as0, ad0)
    esn0, edn0, cv0 = _prep_scores(es0, ed0)
    acc0, den0 = _sc_edge(z0, esn0, edn0, srcl, dstl, cv0)
    a0A, a0B, d0A, d0B = _split_heads(acc0, den0)

    h1, z1, es1, ed1 = _norm_proj(a0A, a0B, d0A, d0B, W1, as1, ad1)
    esn1, edn1, cv1 = _prep_scores(es1, ed1)
    acc1, den1 = _sc_edge(z1, esn1, edn1, srcl, dstl, cv1)
    a1A, a1B, d1A, d1B = _split_heads(acc1, den1)

    return _gru_out(h1, a1A, a1B, d1A, d1B, gWih, gWhh, gbih, gbhh, out_W)


# speedup vs baseline: 36.1572x; 1.0154x over previous
import functools

import jax
import jax.numpy as jnp
from jax import lax
from jax.experimental import pallas as pl
from jax.experimental.pallas import tpu as pltpu
from jax.experimental.pallas import tpu_sc as plsc

N = 10000
E = 160000
IN = 256
H = 128
HEADS = 2
HID = 256
OUT = 256

NP = 10240
CHUNK = 112
NCHUNK = 90
EPAD = 16 * NCHUNK * CHUNK

_B = 2000


def _dotT(a, b):
    return lax.dot_general(a, b, (((1,), (1,)), ((), ())),
                           preferred_element_type=jnp.float32)


def _score_accum(h, i, es, ed, esm_ref, edm_ref, cv_ref):
    @pl.when(jnp.logical_and(h == 0, i == 0))
    def _():
        for hh in range(HEADS):
            esm_ref[hh] = -1e30
            edm_ref[hh] = -1e30

    esm_ref[h] = jnp.maximum(esm_ref[h], jnp.max(es))
    edm_ref[h] = jnp.maximum(edm_ref[h], jnp.max(ed))
    c2 = jnp.stack([jnp.maximum(esm_ref[hh] + edm_ref[hh], 0.0)
                    for hh in range(HEADS)])
    cv_ref[...] = jnp.broadcast_to(c2[:, None], (HEADS, 16))


def _proj_body(hin_ref, W_ref, as_ref, ad_ref, z_ref, es_ref, ed_ref, cv_ref,
               esm_ref, edm_ref):
    h = pl.program_id(0)
    i = pl.program_id(1)
    z = _dotT(hin_ref[...], W_ref[h])
    z_ref[...] = z
    es = z @ as_ref[h]
    ed = z @ ad_ref[h]
    es_ref[...] = es[:, None]
    ed_ref[...] = ed[:, None]
    _score_accum(h, i, es, ed, esm_ref, edm_ref, cv_ref)


def _proj(hin, W, a_s, a_d):
    grid = (HEADS, N // _B)
    full = lambda shape: pl.BlockSpec(shape, lambda h, i: (0,) * len(shape))
    return pl.pallas_call(
        _proj_body,
        grid=grid,
        in_specs=[
            pl.BlockSpec((_B, IN), lambda h, i: (i, 0)),
            full((HEADS, H, IN)),
            full((HEADS, H)),
            full((HEADS, H)),
        ],
        out_specs=[
            pl.BlockSpec((_B, H), lambda h, i: (h * (N // _B) + i, 0)),
            pl.BlockSpec((_B, 1), lambda h, i: (h * (N // _B) + i, 0)),
            pl.BlockSpec((_B, 1), lambda h, i: (h * (N // _B) + i, 0)),
            full((HEADS, 16)),
        ],
        out_shape=[
            jax.ShapeDtypeStruct((HEADS * N, H), jnp.float32),
            jax.ShapeDtypeStruct((HEADS * N, 1), jnp.float32),
            jax.ShapeDtypeStruct((HEADS * N, 1), jnp.float32),
            jax.ShapeDtypeStruct((HEADS, 16), jnp.float32),
        ],
        scratch_shapes=[
            pltpu.SMEM((HEADS,), jnp.float32),
            pltpu.SMEM((HEADS,), jnp.float32),
        ],
    )(hin, W, a_s, a_d)


def _normed(accA, accB, denA, denB):
    hA = jax.nn.relu(accA / jnp.maximum(denA, 1e-38))
    hB = jax.nn.relu(accB / jnp.maximum(denB, 1e-38))
    return jnp.concatenate([hA, hB], axis=1)


def _norm_proj_body(accA_ref, accB_ref, denA_ref, denB_ref, W_ref, as_ref,
                    ad_ref, hcat_ref, z_ref, es_ref, ed_ref, cv_ref,
                    esm_ref, edm_ref):
    hcat = _normed(accA_ref[0], accB_ref[0], denA_ref[0], denB_ref[0])
    hcat_ref[...] = hcat
    h = pl.program_id(0)
    i = pl.program_id(1)
    z = _dotT(hcat, W_ref[h])
    z_ref[...] = z
    es = z @ as_ref[h]
    ed = z @ ad_ref[h]
    es_ref[...] = es[:, None]
    ed_ref[...] = ed[:, None]
    _score_accum(h, i, es, ed, esm_ref, edm_ref, cv_ref)


def _norm_proj(acc3, den3, W, a_s, a_d):
    grid = (HEADS, N // _B)
    full = lambda shape: pl.BlockSpec(shape, lambda h, i: (0,) * len(shape))
    return pl.pallas_call(
        _norm_proj_body,
        grid=grid,
        in_specs=[
            pl.BlockSpec((1, _B, H), lambda h, i: (0, i, 0)),
            pl.BlockSpec((1, _B, H), lambda h, i: (1, i, 0)),
            pl.BlockSpec((1, _B, 1), lambda h, i: (0, i, 0)),
            pl.BlockSpec((1, _B, 1), lambda h, i: (1, i, 0)),
            full((HEADS, H, 2 * H)),
            full((HEADS, H)),
            full((HEADS, H)),
        ],
        out_specs=[
            pl.BlockSpec((_B, 2 * H), lambda h, i: (i, 0)),
            pl.BlockSpec((_B, H), lambda h, i: (h * (N // _B) + i, 0)),
            pl.BlockSpec((_B, 1), lambda h, i: (h * (N // _B) + i, 0)),
            pl.BlockSpec((_B, 1), lambda h, i: (h * (N // _B) + i, 0)),
            full((HEADS, 16)),
        ],
        out_shape=[
            jax.ShapeDtypeStruct((N, 2 * H), jnp.float32),
            jax.ShapeDtypeStruct((HEADS * N, H), jnp.float32),
            jax.ShapeDtypeStruct((HEADS * N, 1), jnp.float32),
            jax.ShapeDtypeStruct((HEADS * N, 1), jnp.float32),
            jax.ShapeDtypeStruct((HEADS, 16), jnp.float32),
        ],
        scratch_shapes=[
            pltpu.SMEM((HEADS,), jnp.float32),
            pltpu.SMEM((HEADS,), jnp.float32),
        ],
    )(acc3, acc3, den3, den3, W, a_s, a_d)


def _gru_out_body(h1_ref, accA_ref, accB_ref, denA_ref, denB_ref, gWih_ref,
                  gWhh_ref, gbih_ref, gbhh_ref, outW_ref, o_ref):
    h1 = h1_ref[...]
    h2 = _normed(accA_ref[0], accB_ref[0], denA_ref[0], denB_ref[0])

    def step(xs, h, Wih, Whh, bih, bhh, with_h):
        gi = _dotT(xs, Wih) + bih
        if with_h:
            gh = _dotT(h, Whh) + bhh
        else:
            gh = jnp.broadcast_to(bhh, gi.shape)
        i_r, i_z, i_n = gi[:, :HID], gi[:, HID:2 * HID], gi[:, 2 * HID:]
        h_r, h_z, h_n = gh[:, :HID], gh[:, HID:2 * HID], gh[:, 2 * HID:]
        r = jax.nn.sigmoid(i_r + h_r)
        zg = jax.nn.sigmoid(i_z + h_z)
        n = jnp.tanh(i_n + r * h_n)
        if with_h:
            return (1.0 - zg) * n + zg * h
        return (1.0 - zg) * n

    f0 = step(h1, None, gWih_ref[0], gWhh_ref[0], gbih_ref[0], gbhh_ref[0],
              False)
    f1 = step(h2, f0, gWih_ref[0], gWhh_ref[0], gbih_ref[0], gbhh_ref[0],
              True)
    b0 = step(h2, None, gWih_ref[1], gWhh_ref[1], gbih_ref[1], gbhh_ref[1],
              False)
    b1 = step(h1, b0, gWih_ref[1], gWhh_ref[1], gbih_ref[1], gbhh_ref[1],
              True)

    outW = outW_ref[...]
    o_ref[...] = (_dotT(f0 + f1, outW[:, :HID]) +
                  _dotT(b0 + b1, outW[:, HID:]))


def _gru_out(h1, acc3, den3, gWih, gWhh, gbih, gbhh, out_W):
    grid = (N // _B,)
    full = lambda shape: pl.BlockSpec(shape, lambda i: (0,) * len(shape))
    return pl.pallas_call(
        _gru_out_body,
        grid=grid,
        in_specs=[
            pl.BlockSpec((_B, HID), lambda i: (i, 0)),
            pl.BlockSpec((1, _B, H), lambda i: (0, i, 0)),
            pl.BlockSpec((1, _B, H), lambda i: (1, i, 0)),
            pl.BlockSpec((1, _B, 1), lambda i: (0, i, 0)),
            pl.BlockSpec((1, _B, 1), lambda i: (1, i, 0)),
            full((2, 3 * HID, HID)),
            full((2, 3 * HID, HID)),
            full((2, 3 * HID)),
            full((2, 3 * HID)),
            full((OUT, 2 * HID)),
        ],
        out_specs=pl.BlockSpec((_B, OUT), lambda i: (i, 0)),
        out_shape=jax.ShapeDtypeStruct((N, OUT), jnp.float32),
    )(h1, acc3, acc3, den3, den3, gWih, gWhh, gbih, gbhh, out_W)


def _sc_edge_body(z_hbm, esn_hbm, edn_hbm, srcl_hbm, dstl_hbm, cv_hbm,
                  acc_hbm, den_hbm,
                  zrows0, src0, dst0, dstg0, a0, es0, ed0,
                  zrows1, src1, dst1, dstg1, a1, es1, ed1,
                  zrows2, src2, dst2, dstg2, a2, es2, ed2,
                  pss0, psd0, pss1, psd1, pss2, psd2,
                  cv_l, acc_s, den_s,
                  sem_i0, sem_i1, sem_i2,
                  semg0, semg1, semg2, sems0, sems1, sems2):
    c = lax.axis_index("c")
    s = lax.axis_index("s")
    cN = c * N

    bufs = [
        (zrows0, src0, dst0, dstg0, a0, es0, ed0, semg0, sems0),
        (zrows1, src1, dst1, dstg1, a1, es1, ed1, semg1, sems1),
        (zrows2, src2, dst2, dstg2, a2, es2, ed2, semg2, sems2),
    ]
    stages = [(pss0, psd0, sem_i0), (pss1, psd1, sem_i1), (pss2, psd2, sem_i2)]

    zero16 = jnp.zeros((16,), jnp.float32)

    def zero_zrows(r, _):
        for k in range(8):
            zrows0[r, pl.ds(k * 16, 16)] = zero16
        return 0

    lax.fori_loop(0, CHUNK, zero_zrows, 0)
    for i in range(CHUNK // 16):
        a0[pl.ds(i * 16, 16)] = zero16

    row0 = s * 640
    for k in range(5):
        pltpu.sync_copy(zrows0.at[pl.ds(0, 112)],
                        acc_s.at[pl.ds(row0 + 112 * k, 112)])
        pltpu.sync_copy(a0.at[pl.ds(0, 112)],
                        den_s.at[pl.ds(row0 + 112 * k, 112)])
    pltpu.sync_copy(zrows0.at[pl.ds(0, 80)], acc_s.at[pl.ds(row0 + 560, 80)])
    pltpu.sync_copy(a0.at[pl.ds(0, 80)], den_s.at[pl.ds(row0 + 560, 80)])

    pltpu.sync_copy(cv_hbm.at[pl.ds(c * 16, 16)], cv_l)

    plsc.subcore_barrier()

    cv = cv_l[...]


    def fire_idx(j, stage):
        ssrc, sdst, sip = stage
        row = j * 16 + s
        pltpu.async_copy(srcl_hbm.at[pl.ds(row, 1)], ssrc, sip)
        pltpu.async_copy(dstl_hbm.at[pl.ds(row, 1)], sdst, sip)

    def prep(j, B, stage, in_loop):
        zr, sv, dv, dg, av, ec, dc, sg, ss = B
        ssrc, sdst, sip = stage
        row = j * 16 + s
        pltpu.make_async_copy(srcl_hbm.at[pl.ds(row, 1)], ssrc, sip).wait()
        pltpu.make_async_copy(dstl_hbm.at[pl.ds(row, 1)], sdst, sip).wait()
        for i in range(CHUNK // 16):
            off = i * 16
            sv[0, pl.ds(off, 16)] = ssrc[0, pl.ds(off, 16)] + cN
            dvv = sdst[0, pl.ds(off, 16)]
            dv[0, pl.ds(off, 16)] = dvv
            dg[0, pl.ds(off, 16)] = jnp.minimum(dvv, N - 1) + cN

        if in_loop:
            @pl.when(j + 3 < NCHUNK)
            def _():
                fire_idx(j + 3, stage)
        else:
            fire_idx(j + 3, stage)

        pltpu.async_copy(esn_hbm.at[sv.at[0]], ec, sg)
        pltpu.async_copy(edn_hbm.at[dg.at[0]], dc, sg)
        pltpu.async_copy(z_hbm.at[sv.at[0]], zr, sg)

    def drain_scatters(B):
        zr, sv, dv, dg, av, ec, dc, sg, ss = B
        pltpu.make_async_copy(av, den_s.at[dv.at[0]], ss).wait()
        pltpu.make_async_copy(zr, acc_s.at[dv.at[0]], ss).wait()

    def compute(B):
        zr, sv, dv, dg, av, ec, dc, sg, ss = B
        pltpu.make_async_copy(esn_hbm.at[sv.at[0]], ec, sg).wait()
        pltpu.make_async_copy(edn_hbm.at[dg.at[0]], dc, sg).wait()
        pltpu.make_async_copy(z_hbm.at[sv.at[0]], zr, sg).wait()

        for i in range(CHUNK // 16):
            off = i * 16
            e = ec[pl.ds(off, 16)] + dc[pl.ds(off, 16)]
            e = jnp.where(e > 0, e, 0.2 * e)
            av[pl.ds(off, 16)] = jnp.exp(e - cv)

        def scale_rows16(t, _):
            a16 = av[pl.ds(t * 16, 16)]
            for jj in range(16):
                r = t * 16 + jj
                a_sc = a16[jj]
                for k in range(8):
                    zr[r, pl.ds(k * 16, 16)] = zr[r, pl.ds(k * 16, 16)] * a_sc
            return 0

        lax.fori_loop(0, CHUNK // 16, scale_rows16, 0)

        pltpu.async_copy(av, den_s.at[dv.at[0]], ss, add=True)
        pltpu.async_copy(zr, acc_s.at[dv.at[0]], ss, add=True)

    fire_idx(0, stages[0])
    fire_idx(1, stages[1])
    fire_idx(2, stages[2])
    prep(0, bufs[0], stages[0], False)
    prep(1, bufs[1], stages[1], False)

    def group_body(i, _):
        for k in range(3):
            g = 3 * i + k
            j = g + 2
            q = (k + 2) % 3
            B_next = bufs[q]

            @pl.when(j < NCHUNK)
            def _():
                @pl.when(j >= 3)
                def _():
                    drain_scatters(B_next)
                prep(j, B_next, stages[q], True)

            compute(bufs[k])
        return 0

    lax.fori_loop(0, NCHUNK // 3, group_body, 0)

    for k in range(3):
        drain_scatters(bufs[k])

    plsc.subcore_barrier()

    pltpu.sync_copy(acc_s.at[pl.ds(row0, 640)],
                    acc_hbm.at[c, pl.ds(row0, 640)])
    pltpu.sync_copy(den_s.at[pl.ds(row0, 640)],
                    den_hbm.at[pl.ds(c * NP + row0, 640)])


def _sc_edge(z, esn, edn, srcl, dstl, cvec):
    mesh = plsc.VectorSubcoreMesh(core_axis_name="c", subcore_axis_name="s")
    buf_set = [
        pltpu.VMEM((CHUNK, H), jnp.float32),
        pltpu.VMEM((1, CHUNK), jnp.int32),
        pltpu.VMEM((1, CHUNK), jnp.int32),
        pltpu.VMEM((1, CHUNK), jnp.int32),
        pltpu.VMEM((CHUNK,), jnp.float32),
        pltpu.VMEM((CHUNK,), jnp.float32),
        pltpu.VMEM((CHUNK,), jnp.float32),
    ]
    fn = pl.kernel(
        _sc_edge_body,
        mesh=mesh,
        compiler_params=pltpu.CompilerParams(needs_layout_passes=False),
        out_type=[
            jax.ShapeDtypeStruct((HEADS, NP, H), jnp.float32),
            jax.ShapeDtypeStruct((HEADS * NP,), jnp.float32),
        ],
        scratch_types=(
            buf_set * 3
            + [pltpu.VMEM((1, CHUNK), jnp.int32)] * 6
            + [
                pltpu.VMEM((16,), jnp.float32),
                pltpu.VMEM_SHARED((NP, H), jnp.float32),
                pltpu.VMEM_SHARED((NP,), jnp.float32),
            ]
            + [pltpu.SemaphoreType.DMA] * 9
        ),
    )
    return fn(z, esn, edn, srcl, dstl, cvec)


def kernel(x, edge_index, W0, as0, ad0, W1, as1, ad1, gWih, gWhh, gbih, gbhh,
           s_w, out_W):
    src = edge_index[0]
    dst = edge_index[1]
    srcl = jnp.pad(src, (0, EPAD - E)).reshape(-1, CHUNK)
    dstl = jnp.pad(dst, (0, EPAD - E), constant_values=N).reshape(-1, CHUNK)

    z0, es0, ed0, cv0 = _proj(x, W0, as0, ad0)
    acc0, den0 = _sc_edge(z0, es0.reshape(-1), ed0.reshape(-1), srcl, dstl,
                          cv0.reshape(-1))
    den0 = den0.reshape(HEADS, NP, 1)

    h1, z1, es1, ed1, cv1 = _norm_proj(acc0, den0, W1, as1, ad1)
    acc1, den1 = _sc_edge(z1, es1.reshape(-1), ed1.reshape(-1), srcl, dstl,
                          cv1.reshape(-1))
    den1 = den1.reshape(HEADS, NP, 1)

    return _gru_out(h1, acc1, den1, gWih, gWhh, gbih, gbhh, out_W)
